# baseline (speedup 1.0000x reference)
import functools

import jax
import jax.numpy as jnp
from jax import lax
from jax.experimental import pallas as pl
from jax.experimental.pallas import tpu as pltpu

E = 8
TOPK = 2
N = 2048
D = 1024
I_ = 1024
SI = 1024
OUT = 1024
ROUTE_SCALE = 1.0

BLK = 128
NB = (TOPK * N + E * (BLK - 1) + BLK - 1) // BLK
P = NB * BLK
GCH = 512
CBLK = 128
SBLK = 256


def _routing_body(x_ref, gw_ref, w_ref, pos_ref, g_ref, be_ref, bv_ref):
    x = x_ref[...]
    gw = gw_ref[...]
    s = lax.dot_general(x, gw, (((1,), (1,)), ((), ())),
                        preferred_element_type=jnp.float32)
    m = jnp.max(s, axis=1, keepdims=True)
    p = jnp.exp(s - m)
    p = p / jnp.sum(p, axis=1, keepdims=True)

    iota_e = lax.broadcasted_iota(jnp.int32, (N, E), 1)
    v1 = jnp.max(p, axis=1, keepdims=True)
    i1 = jnp.min(jnp.where(p == v1, iota_e, E), axis=1, keepdims=True)
    pm = jnp.where(iota_e == i1, -1.0, p)
    v2 = jnp.max(pm, axis=1, keepdims=True)
    i2 = jnp.min(jnp.where(pm == v2, iota_e, E), axis=1, keepdims=True)

    oh = (iota_e == i1).astype(jnp.float32) + (iota_e == i2).astype(jnp.float32)
    c = jnp.sum(oh, axis=0, keepdims=True)
    ssz = jnp.floor((c + (BLK - 1)) * (1.0 / BLK)) * BLK
    u8s = (lax.broadcasted_iota(jnp.int32, (E, E), 0) <
           lax.broadcasted_iota(jnp.int32, (E, E), 1)).astype(jnp.float32)
    off = lax.dot_general(ssz, u8s, (((1,), (0,)), ((), ())),
                          precision=lax.Precision.HIGHEST,
                          preferred_element_type=jnp.float32)

    bstart = lax.broadcasted_iota(jnp.int32, (1, NB), 1).astype(jnp.float32) * BLK
    be = jnp.full((1, NB), float(E - 1), jnp.float32)
    for e in range(E):
        off_e = off[0:1, e:e + 1]
        end_e = off_e + ssz[0:1, e:e + 1]
        be = jnp.where((bstart >= off_e) & (bstart < end_e), float(e), be)
    total = off[0:1, E - 1:E] + ssz[0:1, E - 1:E]
    bv = (bstart < total)
    be_ref[...] = be.astype(jnp.int32)
    bv_ref[...] = bv.astype(jnp.int32)

    lns = (lax.broadcasted_iota(jnp.int32, (N, N), 1) <
           lax.broadcasted_iota(jnp.int32, (N, N), 0)).astype(jnp.float32)
    oh1 = (iota_e == i1).astype(jnp.float32)
    oh2 = (iota_e == i2).astype(jnp.float32)
    ex = lax.dot_general(lns, oh, (((1,), (0,)), ((), ())),
                         precision=lax.Precision.HIGHEST,
                         preferred_element_type=jnp.float32)
    pos0 = jnp.sum(oh1 * (off + ex), axis=1, keepdims=True)
    pos1 = jnp.sum(oh2 * (off + ex + oh1), axis=1, keepdims=True)

    pos_ref[...] = jnp.concatenate([pos0, pos1], axis=1).astype(jnp.int32)
    w_ref[...] = jnp.concatenate([v1, v2], axis=1) * ROUTE_SCALE

    tok = lax.broadcasted_iota(jnp.int32, (1, N), 1).astype(jnp.float32)
    for ci in range(P // GCH):
        lane = (lax.broadcasted_iota(jnp.int32, (1, GCH), 1) + ci * GCH
                ).astype(jnp.float32)
        mm = (pos0 == lane).astype(jnp.float32) + (pos1 == lane).astype(jnp.float32)
        gch = lax.dot_general(tok, mm, (((1,), (0,)), ((), ())),
                              precision=lax.Precision.HIGHEST,
                              preferred_element_type=jnp.float32)
        g_ref[ci:ci + 1, :] = gch.astype(jnp.int32)


def _routing(x, gate_w):
    return pl.pallas_call(
        _routing_body,
        out_shape=[
            jax.ShapeDtypeStruct((N, TOPK), jnp.float32),
            jax.ShapeDtypeStruct((N, TOPK), jnp.int32),
            jax.ShapeDtypeStruct((P // GCH, GCH), jnp.int32),
            jax.ShapeDtypeStruct((1, NB), jnp.int32),
            jax.ShapeDtypeStruct((1, NB), jnp.int32),
        ],
    )(x, gate_w)


def _gemm_body(be_ref, bv_ref, g_ref, x_ref, w1_ref, b1_ref, w3_ref, b3_ref,
               w2_ref, b2_ref, eo_ref, xs_ref):
    b = pl.program_id(0)

    @pl.when(bv_ref[b] == 1)
    def _():
        def gather_row(r, carry):
            t = g_ref[b * BLK + r]
            xs_ref[pl.ds(r, 1), :] = x_ref[pl.ds(t, 1), :]
            return carry
        lax.fori_loop(0, BLK, gather_row, 0)
        xs = xs_ref[...]
        h1 = lax.dot_general(xs, w1_ref[0], (((1,), (1,)), ((), ())),
                             preferred_element_type=jnp.float32) + b1_ref[0]
        h3 = lax.dot_general(xs, w3_ref[0], (((1,), (1,)), ((), ())),
                             preferred_element_type=jnp.float32) + b3_ref[0]
        h = jnp.where(h1 >= 0, h1, 0.01 * h1) * h3
        eo_ref[...] = lax.dot_general(h, w2_ref[0], (((1,), (1,)), ((), ())),
                                      preferred_element_type=jnp.float32) + b2_ref[0]

    @pl.when(bv_ref[b] == 0)
    def _():
        eo_ref[...] = jnp.zeros((BLK, D), jnp.float32)


def _grouped_gemm(x, w1, b1, w3, b3, w2, b2, g, be, bv):
    grid_spec = pltpu.PrefetchScalarGridSpec(
        num_scalar_prefetch=3,
        grid=(NB,),
        in_specs=[
            pl.BlockSpec((N, D), lambda b, be, bv, g: (0, 0)),
            pl.BlockSpec((1, I_, D), lambda b, be, bv, g: (be[b], 0, 0)),
            pl.BlockSpec((1, 1, I_), lambda b, be, bv, g: (be[b], 0, 0)),
            pl.BlockSpec((1, I_, D), lambda b, be, bv, g: (be[b], 0, 0)),
            pl.BlockSpec((1, 1, I_), lambda b, be, bv, g: (be[b], 0, 0)),
            pl.BlockSpec((1, D, I_), lambda b, be, bv, g: (be[b], 0, 0)),
            pl.BlockSpec((1, 1, D), lambda b, be, bv, g: (be[b], 0, 0)),
        ],
        out_specs=pl.BlockSpec((BLK, D), lambda b, be, bv, g: (b, 0)),
        scratch_shapes=[pltpu.VMEM((BLK, D), jnp.float32)],
    )
    return pl.pallas_call(
        _gemm_body,
        grid_spec=grid_spec,
        out_shape=jax.ShapeDtypeStruct((P, D), jnp.float32),
    )(be, bv, g, x, w1, b1.reshape(E, 1, I_), w3, b3.reshape(E, 1, I_),
      w2, b2.reshape(E, 1, D))


def _shared_body(x_ref, sw1_ref, sb1_ref, sw3_ref, sb3_ref, sw2_ref, sb2_ref,
                 z_ref):
    x = x_ref[...]
    s1 = lax.dot_general(x, sw1_ref[...], (((1,), (1,)), ((), ())),
                         preferred_element_type=jnp.float32) + sb1_ref[...]
    s3 = lax.dot_general(x, sw3_ref[...], (((1,), (1,)), ((), ())),
                         preferred_element_type=jnp.float32) + sb3_ref[...]
    h = jnp.where(s1 >= 0, s1, 0.01 * s1) * s3
    z_ref[...] = lax.dot_general(h, sw2_ref[...], (((1,), (1,)), ((), ())),
                                 preferred_element_type=jnp.float32) + sb2_ref[...]


def _shared_ffn(x, sw1, sb1, sw3, sb3, sw2, sb2):
    return pl.pallas_call(
        _shared_body,
        grid=(N // SBLK,),
        in_specs=[
            pl.BlockSpec((SBLK, D), lambda b: (b, 0)),
            pl.BlockSpec((SI, D), lambda b: (0, 0)),
            pl.BlockSpec((1, SI), lambda b: (0, 0)),
            pl.BlockSpec((SI, D), lambda b: (0, 0)),
            pl.BlockSpec((1, SI), lambda b: (0, 0)),
            pl.BlockSpec((D, SI), lambda b: (0, 0)),
            pl.BlockSpec((1, D), lambda b: (0, 0)),
        ],
        out_specs=pl.BlockSpec((SBLK, D), lambda b: (b, 0)),
        out_shape=jax.ShapeDtypeStruct((N, D), jnp.float32),
    )(x, sw1, sb1.reshape(1, SI), sw3, sb3.reshape(1, SI), sw2,
      sb2.reshape(1, D))


def _combine_body(pos_ref, eo_ref, z_ref, wt_ref, ow_ref, ob_ref, out_ref,
                  y_ref):
    b = pl.program_id(0)

    def row(r, carry):
        p0 = pos_ref[(b * CBLK + r) * 2]
        p1 = pos_ref[(b * CBLK + r) * 2 + 1]
        w0 = wt_ref[pl.ds(r, 1), 0:1]
        w1v = wt_ref[pl.ds(r, 1), 1:2]
        y_ref[pl.ds(r, 1), :] = (w0 * eo_ref[pl.ds(p0, 1), :] +
                                 w1v * eo_ref[pl.ds(p1, 1), :])
        return carry
    lax.fori_loop(0, CBLK, row, 0)
    yz = y_ref[...] + z_ref[...]
    out_ref[...] = lax.dot_general(yz, ow_ref[...], (((1,), (1,)), ((), ())),
                                   preferred_element_type=jnp.float32) + ob_ref[...]


def _combine_out(eo, z, wts, pos_flat, out_w, out_b):
    grid_spec = pltpu.PrefetchScalarGridSpec(
        num_scalar_prefetch=1,
        grid=(N // CBLK,),
        in_specs=[
            pl.BlockSpec((P, D), lambda b, pos: (0, 0)),
            pl.BlockSpec((CBLK, D), lambda b, pos: (b, 0)),
            pl.BlockSpec((CBLK, TOPK), lambda b, pos: (b, 0)),
            pl.BlockSpec((OUT, D), lambda b, pos: (0, 0)),
            pl.BlockSpec((1, OUT), lambda b, pos: (0, 0)),
        ],
        out_specs=pl.BlockSpec((CBLK, OUT), lambda b, pos: (b, 0)),
        scratch_shapes=[pltpu.VMEM((CBLK, D), jnp.float32)],
    )
    return pl.pallas_call(
        _combine_body,
        grid_spec=grid_spec,
        out_shape=jax.ShapeDtypeStruct((N, OUT), jnp.float32),
    )(pos_flat, eo, z, wts, out_w, out_b.reshape(1, OUT))


@jax.jit
def _moe(x, gate_w, w1, b1, w2, b2, w3, b3, sw1, sb1, sw2, sb2, sw3, sb3,
         out_w, out_b):
    wts, pos, g, be, bv = _routing(x, gate_w)
    z = _shared_ffn(x, sw1, sb1, sw3, sb3, sw2, sb2)
    eo = _grouped_gemm(x, w1, b1, w3, b3, w2, b2,
                       g.reshape(P), be.reshape(NB), bv.reshape(NB))
    return _combine_out(eo, z, wts, pos.reshape(TOPK * N), out_w, out_b)


def kernel(x, task_id, gate_w, W1, B1, W2, B2, W3, B3, sw1, sb1, sw2, sb2,
           sw3, sb3, out_w, out_b):
    xf = x.reshape(N, D)
    return _moe(xf, gate_w, W1, B1, W2, B2, W3, B3, sw1, sb1, sw2, sb2,
                sw3, sb3, out_w, out_b)


# speedup vs baseline: 1.0086x; 1.0086x over previous
import functools

import jax
import jax.numpy as jnp
from jax import lax
from jax.experimental import pallas as pl
from jax.experimental.pallas import tpu as pltpu
from jax.experimental.pallas import tpu_sc as plsc

E = 8
TOPK = 2
N = 2048
D = 1024
I_ = 1024
SI = 1024
OUT = 1024
ROUTE_SCALE = 1.0

BLK = 128
NB = (TOPK * N + E * (BLK - 1) + BLK - 1) // BLK
P = NB * BLK
GCH = 512
SBLK = 256
FBLK = 128

NC = 2
NS = 16
NW = NC * NS
GROWS = P // NW
GCHUNK = 32
CTOK = N // NW
CCH = 16


def _routing_body(x_ref, gw_ref, pos_ref, w_ref, be_ref, bv_ref):
    x = x_ref[...]
    gw = gw_ref[...]
    s = lax.dot_general(x, gw, (((1,), (1,)), ((), ())),
                        preferred_element_type=jnp.float32)
    m = jnp.max(s, axis=1, keepdims=True)
    p = jnp.exp(s - m)
    p = p / jnp.sum(p, axis=1, keepdims=True)

    iota_e = lax.broadcasted_iota(jnp.int32, (N, E), 1)
    v1 = jnp.max(p, axis=1, keepdims=True)
    i1 = jnp.min(jnp.where(p == v1, iota_e, E), axis=1, keepdims=True)
    pm = jnp.where(iota_e == i1, -1.0, p)
    v2 = jnp.max(pm, axis=1, keepdims=True)
    i2 = jnp.min(jnp.where(pm == v2, iota_e, E), axis=1, keepdims=True)

    oh1 = (iota_e == i1).astype(jnp.float32)
    oh2 = (iota_e == i2).astype(jnp.float32)
    oh = oh1 + oh2
    c = jnp.sum(oh, axis=0, keepdims=True)
    ssz = jnp.floor((c + (BLK - 1)) * (1.0 / BLK)) * BLK
    u8s = (lax.broadcasted_iota(jnp.int32, (E, E), 0) <
           lax.broadcasted_iota(jnp.int32, (E, E), 1)).astype(jnp.float32)
    off = lax.dot_general(ssz, u8s, (((1,), (0,)), ((), ())),
                          precision=lax.Precision.HIGHEST,
                          preferred_element_type=jnp.float32)

    bstart = lax.broadcasted_iota(jnp.int32, (1, NB), 1).astype(jnp.float32) * BLK
    be = jnp.full((1, NB), float(E - 1), jnp.float32)
    for e in range(E):
        off_e = off[0:1, e:e + 1]
        end_e = off_e + ssz[0:1, e:e + 1]
        be = jnp.where((bstart >= off_e) & (bstart < end_e), float(e), be)
    total = off[0:1, E - 1:E] + ssz[0:1, E - 1:E]
    bv = (bstart < total)
    be_ref[...] = be.astype(jnp.int32)
    bv_ref[...] = bv.astype(jnp.int32)

    lns = (lax.broadcasted_iota(jnp.int32, (N, N), 1) <
           lax.broadcasted_iota(jnp.int32, (N, N), 0)).astype(jnp.float32)
    ex = lax.dot_general(lns, oh, (((1,), (0,)), ((), ())),
                         precision=lax.Precision.HIGHEST,
                         preferred_element_type=jnp.float32)
    pos0 = jnp.sum(oh1 * (off + ex), axis=1, keepdims=True)
    pos1 = jnp.sum(oh2 * (off + ex + oh1), axis=1, keepdims=True)

    pos_ref[...] = jnp.concatenate([pos0, pos1], axis=1).astype(jnp.int32)
    w_ref[...] = jnp.concatenate([v1, v2], axis=1) * ROUTE_SCALE


def _routing(x, gate_w):
    return pl.pallas_call(
        _routing_body,
        out_shape=[
            jax.ShapeDtypeStruct((N, TOPK), jnp.int32),
            jax.ShapeDtypeStruct((N, TOPK), jnp.float32),
            jax.ShapeDtypeStruct((1, NB), jnp.int32),
            jax.ShapeDtypeStruct((1, NB), jnp.int32),
        ],
    )(x, gate_w)


def _scatter_body(pos_ref, wts_ref, g_ref, wp_ref):
    ci = pl.program_id(0)
    lane = lax.broadcasted_iota(jnp.int32, (1, GCH), 1) + ci * GCH
    m0 = (pos_ref[:, 0:1] == lane).astype(jnp.float32)
    m1 = (pos_ref[:, 1:2] == lane).astype(jnp.float32)
    tok_col = lax.broadcasted_iota(jnp.int32, (N, 1), 0).astype(jnp.float32)
    gch = lax.dot_general(m0 + m1, tok_col, (((0,), (0,)), ((), ())),
                          precision=lax.Precision.HIGHEST,
                          preferred_element_type=jnp.float32)
    wch = (lax.dot_general(m0, wts_ref[:, 0:1], (((0,), (0,)), ((), ())),
                           precision=lax.Precision.HIGHEST,
                           preferred_element_type=jnp.float32) +
           lax.dot_general(m1, wts_ref[:, 1:2], (((0,), (0,)), ((), ())),
                           precision=lax.Precision.HIGHEST,
                           preferred_element_type=jnp.float32))
    g_ref[...] = gch.astype(jnp.int32)
    wp_ref[...] = wch


def _scatter(pos, wts):
    return pl.pallas_call(
        _scatter_body,
        grid=(P // GCH,),
        in_specs=[
            pl.BlockSpec((N, TOPK), lambda c: (0, 0)),
            pl.BlockSpec((N, TOPK), lambda c: (0, 0)),
        ],
        out_specs=[
            pl.BlockSpec((GCH, 1), lambda c: (c, 0)),
            pl.BlockSpec((GCH, 1), lambda c: (c, 0)),
        ],
        out_shape=[
            jax.ShapeDtypeStruct((P, 1), jnp.int32),
            jax.ShapeDtypeStruct((P, 1), jnp.float32),
        ],
    )(pos, wts)


def _sc_gather(x, g):
    mesh = plsc.VectorSubcoreMesh(core_axis_name="c", subcore_axis_name="s",
                                  num_cores=NC)

    @functools.partial(
        pl.kernel, mesh=mesh,
        out_type=jax.ShapeDtypeStruct((P, D), jnp.float32),
        scratch_types=[
            pltpu.VMEM((GROWS,), jnp.int32),
            pltpu.VMEM((GCHUNK, D), jnp.float32),
            pltpu.SemaphoreType.DMA,
        ],
    )
    def k(x_hbm, g_hbm, out_hbm, idx_v, rows_v, sem):
        wid = lax.axis_index("s") * NC + lax.axis_index("c")
        base = wid * GROWS
        pltpu.sync_copy(g_hbm.at[pl.ds(base, GROWS)], idx_v)
        for c in range(GROWS // GCHUNK):
            pltpu.async_copy(
                x_hbm.at[idx_v.at[pl.ds(c * GCHUNK, GCHUNK)]], rows_v,
                sem).wait()
            pltpu.sync_copy(rows_v, out_hbm.at[pl.ds(base + c * GCHUNK,
                                                     GCHUNK)])

    return k(x, g)


def _gemm_body(be_ref, bv_ref, xs_ref, wp_ref, w1_ref, b1_ref, w3_ref,
               b3_ref, w2_ref, b2_ref, eo_ref):
    b = pl.program_id(0)

    @pl.when(bv_ref[b] == 1)
    def _():
        xs = xs_ref[...]
        h1 = lax.dot_general(xs, w1_ref[0], (((1,), (1,)), ((), ())),
                             preferred_element_type=jnp.float32) + b1_ref[0]
        h3 = lax.dot_general(xs, w3_ref[0], (((1,), (1,)), ((), ())),
                             preferred_element_type=jnp.float32) + b3_ref[0]
        h = jnp.where(h1 >= 0, h1, 0.01 * h1) * h3
        eo = lax.dot_general(h, w2_ref[0], (((1,), (1,)), ((), ())),
                             preferred_element_type=jnp.float32) + b2_ref[0]
        eo_ref[...] = eo * wp_ref[...]

    @pl.when(bv_ref[b] == 0)
    def _():
        eo_ref[...] = jnp.zeros((BLK, D), jnp.float32)


def _grouped_gemm(xs, wp, w1, b1, w3, b3, w2, b2, be, bv):
    grid_spec = pltpu.PrefetchScalarGridSpec(
        num_scalar_prefetch=2,
        grid=(NB,),
        in_specs=[
            pl.BlockSpec((BLK, D), lambda b, be, bv: (b, 0)),
            pl.BlockSpec((BLK, 1), lambda b, be, bv: (b, 0)),
            pl.BlockSpec((1, I_, D), lambda b, be, bv: (be[b], 0, 0)),
            pl.BlockSpec((1, 1, I_), lambda b, be, bv: (be[b], 0, 0)),
            pl.BlockSpec((1, I_, D), lambda b, be, bv: (be[b], 0, 0)),
            pl.BlockSpec((1, 1, I_), lambda b, be, bv: (be[b], 0, 0)),
            pl.BlockSpec((1, D, I_), lambda b, be, bv: (be[b], 0, 0)),
            pl.BlockSpec((1, 1, D), lambda b, be, bv: (be[b], 0, 0)),
        ],
        out_specs=pl.BlockSpec((BLK, D), lambda b, be, bv: (b, 0)),
    )
    return pl.pallas_call(
        _gemm_body,
        grid_spec=grid_spec,
        out_shape=jax.ShapeDtypeStruct((P, D), jnp.float32),
    )(be, bv, xs, wp, w1, b1.reshape(E, 1, I_), w3, b3.reshape(E, 1, I_),
      w2, b2.reshape(E, 1, D))


def _sc_combine(eo, pos_flat):
    mesh = plsc.VectorSubcoreMesh(core_axis_name="c", subcore_axis_name="s",
                                  num_cores=NC)

    @functools.partial(
        pl.kernel, mesh=mesh,
        out_type=jax.ShapeDtypeStruct((N, D), jnp.float32),
        scratch_types=[
            pltpu.VMEM((TOPK * CTOK,), jnp.int32),
            pltpu.VMEM((TOPK * CCH, D), jnp.float32),
            pltpu.VMEM((CCH, D), jnp.float32),
            pltpu.SemaphoreType.DMA,
        ],
    )
    def k(eo_hbm, pos_hbm, out_hbm, idx_v, rows_v, y_v, sem):
        wid = lax.axis_index("s") * NC + lax.axis_index("c")
        base = wid * TOPK * CTOK
        pltpu.sync_copy(pos_hbm.at[pl.ds(base, TOPK * CTOK)], idx_v)
        for c in range(CTOK // CCH):
            pltpu.async_copy(
                eo_hbm.at[idx_v.at[pl.ds(c * TOPK * CCH, TOPK * CCH)]],
                rows_v, sem).wait()

            def pair_add(t, carry):
                def lane_add(j, carry2):
                    y_v[t, pl.ds(j * 16, 16)] = (
                        rows_v[2 * t, pl.ds(j * 16, 16)] +
                        rows_v[2 * t + 1, pl.ds(j * 16, 16)])
                    return carry2
                return lax.fori_loop(0, D // 16, lane_add, carry)
            lax.fori_loop(0, CCH, pair_add, 0)
            pltpu.sync_copy(
                y_v, out_hbm.at[pl.ds(wid * CTOK + c * CCH, CCH)])

    return k(eo, pos_flat)


def _shared_body(x_ref, sw1_ref, sb1_ref, sw3_ref, sb3_ref, sw2_ref, sb2_ref,
                 z_ref):
    x = x_ref[...]
    s1 = lax.dot_general(x, sw1_ref[...], (((1,), (1,)), ((), ())),
                         preferred_element_type=jnp.float32) + sb1_ref[...]
    s3 = lax.dot_general(x, sw3_ref[...], (((1,), (1,)), ((), ())),
                         preferred_element_type=jnp.float32) + sb3_ref[...]
    h = jnp.where(s1 >= 0, s1, 0.01 * s1) * s3
    z_ref[...] = lax.dot_general(h, sw2_ref[...], (((1,), (1,)), ((), ())),
                                 preferred_element_type=jnp.float32) + sb2_ref[...]


def _shared_ffn(x, sw1, sb1, sw3, sb3, sw2, sb2):
    return pl.pallas_call(
        _shared_body,
        grid=(N // SBLK,),
        in_specs=[
            pl.BlockSpec((SBLK, D), lambda b: (b, 0)),
            pl.BlockSpec((SI, D), lambda b: (0, 0)),
            pl.BlockSpec((1, SI), lambda b: (0, 0)),
            pl.BlockSpec((SI, D), lambda b: (0, 0)),
            pl.BlockSpec((1, SI), lambda b: (0, 0)),
            pl.BlockSpec((D, SI), lambda b: (0, 0)),
            pl.BlockSpec((1, D), lambda b: (0, 0)),
        ],
        out_specs=pl.BlockSpec((SBLK, D), lambda b: (b, 0)),
        out_shape=jax.ShapeDtypeStruct((N, D), jnp.float32),
    )(x, sw1, sb1.reshape(1, SI), sw3, sb3.reshape(1, SI), sw2,
      sb2.reshape(1, D))


def _final_body(y_ref, z_ref, ow_ref, ob_ref, out_ref):
    yz = y_ref[...] + z_ref[...]
    out_ref[...] = lax.dot_general(yz, ow_ref[...], (((1,), (1,)), ((), ())),
                                   preferred_element_type=jnp.float32) + ob_ref[...]


def _final(y, z, out_w, out_b):
    return pl.pallas_call(
        _final_body,
        grid=(N // FBLK,),
        in_specs=[
            pl.BlockSpec((FBLK, D), lambda b: (b, 0)),
            pl.BlockSpec((FBLK, D), lambda b: (b, 0)),
            pl.BlockSpec((OUT, D), lambda b: (0, 0)),
            pl.BlockSpec((1, OUT), lambda b: (0, 0)),
        ],
        out_specs=pl.BlockSpec((FBLK, OUT), lambda b: (b, 0)),
        out_shape=jax.ShapeDtypeStruct((N, OUT), jnp.float32),
    )(y, z, out_w, out_b.reshape(1, OUT))


@jax.jit
def _moe(x, gate_w, w1, b1, w2, b2, w3, b3, sw1, sb1, sw2, sb2, sw3, sb3,
         out_w, out_b):
    pos, wts, be, bv = _routing(x, gate_w)
    g, wp = _scatter(pos, wts)
    xs = _sc_gather(x, g.reshape(P))
    z = _shared_ffn(x, sw1, sb1, sw3, sb3, sw2, sb2)
    eo = _grouped_gemm(xs, wp, w1, b1, w3, b3, w2, b2,
                       be.reshape(NB), bv.reshape(NB))
    y = _sc_combine(eo, pos.reshape(TOPK * N))
    return _final(y, z, out_w, out_b)


def kernel(x, task_id, gate_w, W1, B1, W2, B2, W3, B3, sw1, sb1, sw2, sb2,
           sw3, sb3, out_w, out_b):
    xf = x.reshape(N, D)
    return _moe(xf, gate_w, W1, B1, W2, B2, W3, B3, sw1, sb1, sw2, sb2,
                sw3, sb3, out_w, out_b)


# speedup vs baseline: 1.5548x; 1.5415x over previous
import functools

import jax
import jax.numpy as jnp
from jax import lax
from jax.experimental import pallas as pl
from jax.experimental.pallas import tpu as pltpu
from jax.experimental.pallas import tpu_sc as plsc

E = 8
TOPK = 2
N = 2048
D = 1024
I_ = 1024
SI = 1024
OUT = 1024
ROUTE_SCALE = 1.0

BLK = 128
NB = (TOPK * N + E * (BLK - 1) + BLK - 1) // BLK
P = NB * BLK
GCH = 512
SBLK = 256
FBLK = 128

NC = 2
NS = 16
NW = NC * NS
GROWS = P // NW
GCHUNK = 32
CTOK = N // NW
CCH = 16


def _routing_body(x_ref, gw_ref, pos_ref, w_ref, be_ref, bv_ref):
    x = x_ref[...]
    gw = gw_ref[...]
    s = lax.dot_general(x, gw, (((1,), (1,)), ((), ())),
                        preferred_element_type=jnp.float32)
    m = jnp.max(s, axis=1, keepdims=True)
    p = jnp.exp(s - m)
    p = p / jnp.sum(p, axis=1, keepdims=True)

    iota_e = lax.broadcasted_iota(jnp.int32, (N, E), 1)
    v1 = jnp.max(p, axis=1, keepdims=True)
    i1 = jnp.min(jnp.where(p == v1, iota_e, E), axis=1, keepdims=True)
    pm = jnp.where(iota_e == i1, -1.0, p)
    v2 = jnp.max(pm, axis=1, keepdims=True)
    i2 = jnp.min(jnp.where(pm == v2, iota_e, E), axis=1, keepdims=True)

    oh1 = (iota_e == i1).astype(jnp.float32)
    oh2 = (iota_e == i2).astype(jnp.float32)
    oh = oh1 + oh2
    c = jnp.sum(oh, axis=0, keepdims=True)
    ssz = jnp.floor((c + (BLK - 1)) * (1.0 / BLK)) * BLK
    u8s = (lax.broadcasted_iota(jnp.int32, (E, E), 0) <
           lax.broadcasted_iota(jnp.int32, (E, E), 1)).astype(jnp.float32)
    off = lax.dot_general(ssz, u8s, (((1,), (0,)), ((), ())),
                          precision=lax.Precision.HIGHEST,
                          preferred_element_type=jnp.float32)

    bstart = lax.broadcasted_iota(jnp.int32, (1, NB), 1).astype(jnp.float32) * BLK
    be = jnp.full((1, NB), float(E - 1), jnp.float32)
    for e in range(E):
        off_e = off[0:1, e:e + 1]
        end_e = off_e + ssz[0:1, e:e + 1]
        be = jnp.where((bstart >= off_e) & (bstart < end_e), float(e), be)
    total = off[0:1, E - 1:E] + ssz[0:1, E - 1:E]
    bv = (bstart < total)
    be_ref[...] = be.astype(jnp.int32)
    bv_ref[...] = bv.astype(jnp.int32)

    lns = (lax.broadcasted_iota(jnp.int32, (N, N), 1) <
           lax.broadcasted_iota(jnp.int32, (N, N), 0)).astype(jnp.float32)
    ex = lax.dot_general(lns, oh, (((1,), (0,)), ((), ())),
                         precision=lax.Precision.HIGHEST,
                         preferred_element_type=jnp.float32)
    pos0 = jnp.sum(oh1 * (off + ex), axis=1, keepdims=True)
    pos1 = jnp.sum(oh2 * (off + ex + oh1), axis=1, keepdims=True)

    pos_ref[...] = jnp.concatenate([pos0, pos1], axis=1).astype(jnp.int32)
    w_ref[...] = jnp.concatenate([v1, v2], axis=1) * ROUTE_SCALE


def _routing(x, gate_w):
    return pl.pallas_call(
        _routing_body,
        out_shape=[
            jax.ShapeDtypeStruct((N, TOPK), jnp.int32),
            jax.ShapeDtypeStruct((N, TOPK), jnp.float32),
            jax.ShapeDtypeStruct((1, NB), jnp.int32),
            jax.ShapeDtypeStruct((1, NB), jnp.int32),
        ],
    )(x, gate_w)


TCH = 16
NCH = CTOK // TCH


def _sc_dispatch(x, p0, p1, w0, w1):
    mesh = plsc.VectorSubcoreMesh(core_axis_name="c", subcore_axis_name="s",
                                  num_cores=NC)

    @functools.partial(
        pl.kernel, mesh=mesh,
        out_type=[
            jax.ShapeDtypeStruct((P, D), jnp.float32),
            jax.ShapeDtypeStruct((P,), jnp.float32),
        ],
        scratch_types=[
            pltpu.VMEM((CTOK, D), jnp.float32),
            pltpu.VMEM((NCH, TCH), jnp.int32),
            pltpu.VMEM((NCH, TCH), jnp.int32),
            pltpu.VMEM((NCH, TCH), jnp.float32),
            pltpu.VMEM((NCH, TCH), jnp.float32),
            pltpu.SemaphoreType.DMA,
        ],
    )
    def k(x_hbm, p0_hbm, p1_hbm, w0_hbm, w1_hbm, xs_hbm, wp_hbm,
          rows_v, i0_v, i1_v, w0_v, w1_v, sem):
        wid = lax.axis_index("s") * NC + lax.axis_index("c")
        pltpu.sync_copy(x_hbm.at[pl.ds(wid * CTOK, CTOK)], rows_v)
        pltpu.sync_copy(p0_hbm.at[wid], i0_v)
        pltpu.sync_copy(p1_hbm.at[wid], i1_v)
        pltpu.sync_copy(w0_hbm.at[wid], w0_v)
        pltpu.sync_copy(w1_hbm.at[wid], w1_v)
        handles = []
        for c in range(NCH):
            src = rows_v.at[pl.ds(c * TCH, TCH)]
            handles.append(pltpu.async_copy(src, xs_hbm.at[i0_v.at[c]], sem))
            handles.append(pltpu.async_copy(src, xs_hbm.at[i1_v.at[c]], sem))
            handles.append(pltpu.async_copy(w0_v.at[c], wp_hbm.at[i0_v.at[c]],
                                            sem))
            handles.append(pltpu.async_copy(w1_v.at[c], wp_hbm.at[i1_v.at[c]],
                                            sem))
        for h in handles:
            h.wait()

    return k(x, p0, p1, w0, w1)


def _gemm_body(be_ref, bv_ref, xs_ref, wp_ref, w1_ref, b1_ref, w3_ref,
               b3_ref, w2_ref, b2_ref, eo_ref):
    b = pl.program_id(0)

    @pl.when(bv_ref[b] == 1)
    def _():
        xs = xs_ref[...]
        h1 = lax.dot_general(xs, w1_ref[0], (((1,), (1,)), ((), ())),
                             preferred_element_type=jnp.float32) + b1_ref[0]
        h3 = lax.dot_general(xs, w3_ref[0], (((1,), (1,)), ((), ())),
                             preferred_element_type=jnp.float32) + b3_ref[0]
        h = jnp.where(h1 >= 0, h1, 0.01 * h1) * h3
        eo = lax.dot_general(h, w2_ref[0], (((1,), (1,)), ((), ())),
                             preferred_element_type=jnp.float32) + b2_ref[0]
        eo_ref[...] = eo * wp_ref[...]

    @pl.when(bv_ref[b] == 0)
    def _():
        eo_ref[...] = jnp.zeros((BLK, D), jnp.float32)


def _grouped_gemm(xs, wp, w1, b1, w3, b3, w2, b2, be, bv):
    grid_spec = pltpu.PrefetchScalarGridSpec(
        num_scalar_prefetch=2,
        grid=(NB,),
        in_specs=[
            pl.BlockSpec((BLK, D), lambda b, be, bv: (b, 0)),
            pl.BlockSpec((BLK, 1), lambda b, be, bv: (b, 0)),
            pl.BlockSpec((1, I_, D), lambda b, be, bv: (be[b], 0, 0)),
            pl.BlockSpec((1, 1, I_), lambda b, be, bv: (be[b], 0, 0)),
            pl.BlockSpec((1, I_, D), lambda b, be, bv: (be[b], 0, 0)),
            pl.BlockSpec((1, 1, I_), lambda b, be, bv: (be[b], 0, 0)),
            pl.BlockSpec((1, D, I_), lambda b, be, bv: (be[b], 0, 0)),
            pl.BlockSpec((1, 1, D), lambda b, be, bv: (be[b], 0, 0)),
        ],
        out_specs=pl.BlockSpec((BLK, D), lambda b, be, bv: (b, 0)),
    )
    return pl.pallas_call(
        _gemm_body,
        grid_spec=grid_spec,
        out_shape=jax.ShapeDtypeStruct((P, D), jnp.float32),
    )(be, bv, xs, wp, w1, b1.reshape(E, 1, I_), w3, b3.reshape(E, 1, I_),
      w2, b2.reshape(E, 1, D))


def _sc_combine(eo, pos_flat):
    mesh = plsc.VectorSubcoreMesh(core_axis_name="c", subcore_axis_name="s",
                                  num_cores=NC)

    @functools.partial(
        pl.kernel, mesh=mesh,
        out_type=jax.ShapeDtypeStruct((N, D), jnp.float32),
        scratch_types=[
            pltpu.VMEM((TOPK * CTOK,), jnp.int32),
            pltpu.VMEM((TOPK * CCH, D), jnp.float32),
            pltpu.VMEM((TOPK * CCH, D), jnp.float32),
            pltpu.VMEM((CCH, D), jnp.float32),
            pltpu.SemaphoreType.DMA,
            pltpu.SemaphoreType.DMA,
        ],
    )
    def k(eo_hbm, pos_hbm, out_hbm, idx_v, rows_a, rows_b, y_v, sem_a,
          sem_b):
        wid = lax.axis_index("s") * NC + lax.axis_index("c")
        base = wid * TOPK * CTOK
        nch = CTOK // CCH
        pltpu.sync_copy(pos_hbm.at[pl.ds(base, TOPK * CTOK)], idx_v)
        bufs = [rows_a, rows_b]
        sems = [sem_a, sem_b]
        handles = [None] * nch
        handles[0] = pltpu.async_copy(
            eo_hbm.at[idx_v.at[pl.ds(0, TOPK * CCH)]], bufs[0], sems[0])
        for c in range(nch):
            handles[c].wait()
            if c + 1 < nch:
                handles[c + 1] = pltpu.async_copy(
                    eo_hbm.at[idx_v.at[pl.ds((c + 1) * TOPK * CCH,
                                             TOPK * CCH)]],
                    bufs[(c + 1) % 2], sems[(c + 1) % 2])
            rows_v = bufs[c % 2]

            def pair_add(t, carry):
                def lane_add(j, carry2):
                    y_v[t, pl.ds(j * 16, 16)] = (
                        rows_v[2 * t, pl.ds(j * 16, 16)] +
                        rows_v[2 * t + 1, pl.ds(j * 16, 16)])
                    return carry2
                return lax.fori_loop(0, D // 16, lane_add, carry)
            lax.fori_loop(0, CCH, pair_add, 0)
            pltpu.sync_copy(
                y_v, out_hbm.at[pl.ds(wid * CTOK + c * CCH, CCH)])

    return k(eo, pos_flat)


def _shared_body(x_ref, sw1_ref, sb1_ref, sw3_ref, sb3_ref, sw2_ref, sb2_ref,
                 z_ref):
    x = x_ref[...]
    s1 = lax.dot_general(x, sw1_ref[...], (((1,), (1,)), ((), ())),
                         preferred_element_type=jnp.float32) + sb1_ref[...]
    s3 = lax.dot_general(x, sw3_ref[...], (((1,), (1,)), ((), ())),
                         preferred_element_type=jnp.float32) + sb3_ref[...]
    h = jnp.where(s1 >= 0, s1, 0.01 * s1) * s3
    z_ref[...] = lax.dot_general(h, sw2_ref[...], (((1,), (1,)), ((), ())),
                                 preferred_element_type=jnp.float32) + sb2_ref[...]


def _shared_ffn(x, sw1, sb1, sw3, sb3, sw2, sb2):
    return pl.pallas_call(
        _shared_body,
        grid=(N // SBLK,),
        in_specs=[
            pl.BlockSpec((SBLK, D), lambda b: (b, 0)),
            pl.BlockSpec((SI, D), lambda b: (0, 0)),
            pl.BlockSpec((1, SI), lambda b: (0, 0)),
            pl.BlockSpec((SI, D), lambda b: (0, 0)),
            pl.BlockSpec((1, SI), lambda b: (0, 0)),
            pl.BlockSpec((D, SI), lambda b: (0, 0)),
            pl.BlockSpec((1, D), lambda b: (0, 0)),
        ],
        out_specs=pl.BlockSpec((SBLK, D), lambda b: (b, 0)),
        out_shape=jax.ShapeDtypeStruct((N, D), jnp.float32),
    )(x, sw1, sb1.reshape(1, SI), sw3, sb3.reshape(1, SI), sw2,
      sb2.reshape(1, D))


def _final_body(y_ref, z_ref, ow_ref, ob_ref, out_ref):
    yz = y_ref[...] + z_ref[...]
    out_ref[...] = lax.dot_general(yz, ow_ref[...], (((1,), (1,)), ((), ())),
                                   preferred_element_type=jnp.float32) + ob_ref[...]


def _final(y, z, out_w, out_b):
    return pl.pallas_call(
        _final_body,
        grid=(N // FBLK,),
        in_specs=[
            pl.BlockSpec((FBLK, D), lambda b: (b, 0)),
            pl.BlockSpec((FBLK, D), lambda b: (b, 0)),
            pl.BlockSpec((OUT, D), lambda b: (0, 0)),
            pl.BlockSpec((1, OUT), lambda b: (0, 0)),
        ],
        out_specs=pl.BlockSpec((FBLK, OUT), lambda b: (b, 0)),
        out_shape=jax.ShapeDtypeStruct((N, OUT), jnp.float32),
    )(y, z, out_w, out_b.reshape(1, OUT))


@jax.jit
def _moe(x, gate_w, w1, b1, w2, b2, w3, b3, sw1, sb1, sw2, sb2, sw3, sb3,
         out_w, out_b):
    pos, wts, be, bv = _routing(x, gate_w)
    xs, wp = _sc_dispatch(
        x,
        pos[:, 0].reshape(NW, NCH, TCH), pos[:, 1].reshape(NW, NCH, TCH),
        wts[:, 0].reshape(NW, NCH, TCH), wts[:, 1].reshape(NW, NCH, TCH))
    z = _shared_ffn(x, sw1, sb1, sw3, sb3, sw2, sb2)
    eo = _grouped_gemm(xs, wp.reshape(P, 1), w1, b1, w3, b3, w2, b2,
                       be.reshape(NB), bv.reshape(NB))
    y = _sc_combine(eo, pos.reshape(TOPK * N))
    return _final(y, z, out_w, out_b)


def kernel(x, task_id, gate_w, W1, B1, W2, B2, W3, B3, sw1, sb1, sw2, sb2,
           sw3, sb3, out_w, out_b):
    xf = x.reshape(N, D)
    return _moe(xf, gate_w, W1, B1, W2, B2, W3, B3, sw1, sb1, sw2, sb2,
                sw3, sb3, out_w, out_b)


# speedup vs baseline: 1.6425x; 1.0564x over previous
import functools

import jax
import jax.numpy as jnp
from jax import lax
from jax.experimental import pallas as pl
from jax.experimental.pallas import tpu as pltpu
from jax.experimental.pallas import tpu_sc as plsc

E = 8
TOPK = 2
N = 2048
D = 1024
I_ = 1024
SI = 1024
OUT = 1024
ROUTE_SCALE = 1.0

BLK = 128
NB = (TOPK * N + E * (BLK - 1) + BLK - 1) // BLK
P = NB * BLK
GCH = 512
SBLK = 256
FBLK = 128

NC = 2
NS = 16
NW = NC * NS
GROWS = P // NW
GCHUNK = 32
CTOK = N // NW
CCH = 16


def _routing_body(x_ref, gw_ref, pos_ref, w_ref, be_ref, bv_ref):
    x = x_ref[...]
    gw = gw_ref[...]
    s = lax.dot_general(x, gw, (((1,), (1,)), ((), ())),
                        preferred_element_type=jnp.float32)
    m = jnp.max(s, axis=1, keepdims=True)
    p = jnp.exp(s - m)
    p = p / jnp.sum(p, axis=1, keepdims=True)

    iota_e = lax.broadcasted_iota(jnp.int32, (N, E), 1)
    v1 = jnp.max(p, axis=1, keepdims=True)
    i1 = jnp.min(jnp.where(p == v1, iota_e, E), axis=1, keepdims=True)
    pm = jnp.where(iota_e == i1, -1.0, p)
    v2 = jnp.max(pm, axis=1, keepdims=True)
    i2 = jnp.min(jnp.where(pm == v2, iota_e, E), axis=1, keepdims=True)

    oh1 = (iota_e == i1).astype(jnp.float32)
    oh2 = (iota_e == i2).astype(jnp.float32)
    oh = oh1 + oh2
    c = jnp.sum(oh, axis=0, keepdims=True)
    ssz = jnp.floor((c + (BLK - 1)) * (1.0 / BLK)) * BLK
    u8s = (lax.broadcasted_iota(jnp.int32, (E, E), 0) <
           lax.broadcasted_iota(jnp.int32, (E, E), 1)).astype(jnp.float32)
    off = lax.dot_general(ssz, u8s, (((1,), (0,)), ((), ())),
                          precision=lax.Precision.HIGHEST,
                          preferred_element_type=jnp.float32)

    bstart = lax.broadcasted_iota(jnp.int32, (1, NB), 1).astype(jnp.float32) * BLK
    be = jnp.full((1, NB), float(E - 1), jnp.float32)
    for e in range(E):
        off_e = off[0:1, e:e + 1]
        end_e = off_e + ssz[0:1, e:e + 1]
        be = jnp.where((bstart >= off_e) & (bstart < end_e), float(e), be)
    total = off[0:1, E - 1:E] + ssz[0:1, E - 1:E]
    bv = (bstart < total)
    be_ref[...] = be.astype(jnp.int32)
    bv_ref[...] = bv.astype(jnp.int32)

    lns = (lax.broadcasted_iota(jnp.int32, (N, N), 1) <
           lax.broadcasted_iota(jnp.int32, (N, N), 0)).astype(jnp.float32)
    ex = lax.dot_general(lns, oh, (((1,), (0,)), ((), ())),
                         preferred_element_type=jnp.float32)
    pos0 = jnp.sum(oh1 * (off + ex), axis=1, keepdims=True)
    pos1 = jnp.sum(oh2 * (off + ex + oh1), axis=1, keepdims=True)

    pos_ref[...] = jnp.concatenate([pos0, pos1], axis=1).astype(jnp.int32)
    w_ref[...] = jnp.concatenate([v1, v2], axis=1) * ROUTE_SCALE


def _routing(x, gate_w):
    return pl.pallas_call(
        _routing_body,
        out_shape=[
            jax.ShapeDtypeStruct((N, TOPK), jnp.int32),
            jax.ShapeDtypeStruct((N, TOPK), jnp.float32),
            jax.ShapeDtypeStruct((1, NB), jnp.int32),
            jax.ShapeDtypeStruct((1, NB), jnp.int32),
        ],
    )(x, gate_w)


TCH = 16
NCH = CTOK // TCH


def _sc_dispatch(x, p0, p1, w0, w1):
    mesh = plsc.VectorSubcoreMesh(core_axis_name="c", subcore_axis_name="s",
                                  num_cores=NC)

    @functools.partial(
        pl.kernel, mesh=mesh,
        out_type=[
            jax.ShapeDtypeStruct((P, D), jnp.float32),
            jax.ShapeDtypeStruct((P,), jnp.float32),
        ],
        scratch_types=[
            pltpu.VMEM((CTOK, D), jnp.float32),
            pltpu.VMEM((NCH, TCH), jnp.int32),
            pltpu.VMEM((NCH, TCH), jnp.int32),
            pltpu.VMEM((NCH, TCH), jnp.float32),
            pltpu.VMEM((NCH, TCH), jnp.float32),
            pltpu.SemaphoreType.DMA,
            pltpu.SemaphoreType.DMA,
            pltpu.SemaphoreType.DMA,
            pltpu.SemaphoreType.DMA,
            pltpu.SemaphoreType.DMA,
        ],
    )
    def k(x_hbm, p0_hbm, p1_hbm, w0_hbm, w1_hbm, xs_hbm, wp_hbm,
          rows_v, i0_v, i1_v, w0_v, w1_v, sem, *sem_x):
        wid = lax.axis_index("s") * NC + lax.axis_index("c")
        xh = [pltpu.async_copy(
            x_hbm.at[pl.ds(wid * CTOK + c * TCH, TCH)],
            rows_v.at[pl.ds(c * TCH, TCH)], sem_x[c]) for c in range(NCH)]
        pltpu.sync_copy(p0_hbm.at[wid], i0_v)
        pltpu.sync_copy(p1_hbm.at[wid], i1_v)
        pltpu.sync_copy(w0_hbm.at[wid], w0_v)
        pltpu.sync_copy(w1_hbm.at[wid], w1_v)
        handles = []
        for c in range(NCH):
            xh[c].wait()
            src = rows_v.at[pl.ds(c * TCH, TCH)]
            handles.append(pltpu.async_copy(src, xs_hbm.at[i0_v.at[c]], sem))
            handles.append(pltpu.async_copy(src, xs_hbm.at[i1_v.at[c]], sem))
            handles.append(pltpu.async_copy(w0_v.at[c], wp_hbm.at[i0_v.at[c]],
                                            sem))
            handles.append(pltpu.async_copy(w1_v.at[c], wp_hbm.at[i1_v.at[c]],
                                            sem))
        for h in handles:
            h.wait()

    return k(x, p0, p1, w0, w1)


def _gemm_body(be_ref, bv_ref, xs_ref, wp_ref, w1_ref, b1_ref, w3_ref,
               b3_ref, w2_ref, b2_ref, eo_ref):
    b = pl.program_id(0)

    @pl.when(bv_ref[b] == 1)
    def _():
        xs = xs_ref[...]
        h1 = lax.dot_general(xs, w1_ref[0], (((1,), (1,)), ((), ())),
                             preferred_element_type=jnp.float32) + b1_ref[0]
        h3 = lax.dot_general(xs, w3_ref[0], (((1,), (1,)), ((), ())),
                             preferred_element_type=jnp.float32) + b3_ref[0]
        h = jnp.where(h1 >= 0, h1, 0.01 * h1) * h3
        eo = lax.dot_general(h, w2_ref[0], (((1,), (1,)), ((), ())),
                             preferred_element_type=jnp.float32) + b2_ref[0]
        eo_ref[...] = eo * wp_ref[...]

    @pl.when(bv_ref[b] == 0)
    def _():
        eo_ref[...] = jnp.zeros((BLK, D), jnp.float32)


def _grouped_gemm(xs, wp, w1, b1, w3, b3, w2, b2, be, bv):
    grid_spec = pltpu.PrefetchScalarGridSpec(
        num_scalar_prefetch=2,
        grid=(NB,),
        in_specs=[
            pl.BlockSpec((BLK, D), lambda b, be, bv: (b, 0)),
            pl.BlockSpec((BLK, 1), lambda b, be, bv: (b, 0)),
            pl.BlockSpec((1, I_, D), lambda b, be, bv: (be[b], 0, 0)),
            pl.BlockSpec((1, 1, I_), lambda b, be, bv: (be[b], 0, 0)),
            pl.BlockSpec((1, I_, D), lambda b, be, bv: (be[b], 0, 0)),
            pl.BlockSpec((1, 1, I_), lambda b, be, bv: (be[b], 0, 0)),
            pl.BlockSpec((1, D, I_), lambda b, be, bv: (be[b], 0, 0)),
            pl.BlockSpec((1, 1, D), lambda b, be, bv: (be[b], 0, 0)),
        ],
        out_specs=pl.BlockSpec((BLK, D), lambda b, be, bv: (b, 0)),
    )
    return pl.pallas_call(
        _gemm_body,
        grid_spec=grid_spec,
        out_shape=jax.ShapeDtypeStruct((P, D), jnp.float32),
    )(be, bv, xs, wp, w1, b1.reshape(E, 1, I_), w3, b3.reshape(E, 1, I_),
      w2, b2.reshape(E, 1, D))


def _sc_combine(eo, pos_flat):
    mesh = plsc.VectorSubcoreMesh(core_axis_name="c", subcore_axis_name="s",
                                  num_cores=NC)

    @functools.partial(
        pl.kernel, mesh=mesh,
        out_type=jax.ShapeDtypeStruct((N, D), jnp.float32),
        scratch_types=[
            pltpu.VMEM((TOPK * CTOK,), jnp.int32),
            pltpu.VMEM((TOPK * CCH, D), jnp.float32),
            pltpu.VMEM((TOPK * CCH, D), jnp.float32),
            pltpu.VMEM((CCH, D), jnp.float32),
            pltpu.VMEM((CCH, D), jnp.float32),
            pltpu.SemaphoreType.DMA,
            pltpu.SemaphoreType.DMA,
            pltpu.SemaphoreType.DMA,
            pltpu.SemaphoreType.DMA,
        ],
    )
    def k(eo_hbm, pos_hbm, out_hbm, idx_v, rows_a, rows_b, y_a, y_b,
          sem_a, sem_b, sem_ya, sem_yb):
        wid = lax.axis_index("s") * NC + lax.axis_index("c")
        base = wid * TOPK * CTOK
        nch = CTOK // CCH
        pltpu.sync_copy(pos_hbm.at[pl.ds(base, TOPK * CTOK)], idx_v)
        bufs = [rows_a, rows_b]
        sems = [sem_a, sem_b]
        ybufs = [y_a, y_b]
        ysems = [sem_ya, sem_yb]
        handles = [None] * nch
        yhandles = [None] * nch
        handles[0] = pltpu.async_copy(
            eo_hbm.at[idx_v.at[pl.ds(0, TOPK * CCH)]], bufs[0], sems[0])
        for c in range(nch):
            handles[c].wait()
            if c + 1 < nch:
                handles[c + 1] = pltpu.async_copy(
                    eo_hbm.at[idx_v.at[pl.ds((c + 1) * TOPK * CCH,
                                             TOPK * CCH)]],
                    bufs[(c + 1) % 2], sems[(c + 1) % 2])
            rows_v = bufs[c % 2]
            y_v = ybufs[c % 2]
            if c >= 2:
                yhandles[c - 2].wait()

            def pair_add(t, carry):
                def lane_add(j, carry2):
                    y_v[t, pl.ds(j * 16, 16)] = (
                        rows_v[2 * t, pl.ds(j * 16, 16)] +
                        rows_v[2 * t + 1, pl.ds(j * 16, 16)])
                    return carry2
                return lax.fori_loop(0, D // 16, lane_add, carry)
            lax.fori_loop(0, CCH, pair_add, 0)
            yhandles[c] = pltpu.async_copy(
                y_v, out_hbm.at[pl.ds(wid * CTOK + c * CCH, CCH)],
                ysems[c % 2])
        for c in range(max(0, nch - 2), nch):
            yhandles[c].wait()

    return k(eo, pos_flat)


def _shared_body(x_ref, sw1_ref, sb1_ref, sw3_ref, sb3_ref, sw2_ref, sb2_ref,
                 z_ref):
    x = x_ref[...]
    s1 = lax.dot_general(x, sw1_ref[...], (((1,), (1,)), ((), ())),
                         preferred_element_type=jnp.float32) + sb1_ref[...]
    s3 = lax.dot_general(x, sw3_ref[...], (((1,), (1,)), ((), ())),
                         preferred_element_type=jnp.float32) + sb3_ref[...]
    h = jnp.where(s1 >= 0, s1, 0.01 * s1) * s3
    z_ref[...] = lax.dot_general(h, sw2_ref[...], (((1,), (1,)), ((), ())),
                                 preferred_element_type=jnp.float32) + sb2_ref[...]


def _shared_ffn(x, sw1, sb1, sw3, sb3, sw2, sb2):
    return pl.pallas_call(
        _shared_body,
        grid=(N // SBLK,),
        in_specs=[
            pl.BlockSpec((SBLK, D), lambda b: (b, 0)),
            pl.BlockSpec((SI, D), lambda b: (0, 0)),
            pl.BlockSpec((1, SI), lambda b: (0, 0)),
            pl.BlockSpec((SI, D), lambda b: (0, 0)),
            pl.BlockSpec((1, SI), lambda b: (0, 0)),
            pl.BlockSpec((D, SI), lambda b: (0, 0)),
            pl.BlockSpec((1, D), lambda b: (0, 0)),
        ],
        out_specs=pl.BlockSpec((SBLK, D), lambda b: (b, 0)),
        out_shape=jax.ShapeDtypeStruct((N, D), jnp.float32),
    )(x, sw1, sb1.reshape(1, SI), sw3, sb3.reshape(1, SI), sw2,
      sb2.reshape(1, D))


def _final_body(y_ref, z_ref, ow_ref, ob_ref, out_ref):
    yz = y_ref[...] + z_ref[...]
    out_ref[...] = lax.dot_general(yz, ow_ref[...], (((1,), (1,)), ((), ())),
                                   preferred_element_type=jnp.float32) + ob_ref[...]


def _final(y, z, out_w, out_b):
    return pl.pallas_call(
        _final_body,
        grid=(N // FBLK,),
        in_specs=[
            pl.BlockSpec((FBLK, D), lambda b: (b, 0)),
            pl.BlockSpec((FBLK, D), lambda b: (b, 0)),
            pl.BlockSpec((OUT, D), lambda b: (0, 0)),
            pl.BlockSpec((1, OUT), lambda b: (0, 0)),
        ],
        out_specs=pl.BlockSpec((FBLK, OUT), lambda b: (b, 0)),
        out_shape=jax.ShapeDtypeStruct((N, OUT), jnp.float32),
    )(y, z, out_w, out_b.reshape(1, OUT))


@jax.jit
def _moe(x, gate_w, w1, b1, w2, b2, w3, b3, sw1, sb1, sw2, sb2, sw3, sb3,
         out_w, out_b):
    pos, wts, be, bv = _routing(x, gate_w)
    xs, wp = _sc_dispatch(
        x,
        pos[:, 0].reshape(NW, NCH, TCH), pos[:, 1].reshape(NW, NCH, TCH),
        wts[:, 0].reshape(NW, NCH, TCH), wts[:, 1].reshape(NW, NCH, TCH))
    z = _shared_ffn(x, sw1, sb1, sw3, sb3, sw2, sb2)
    eo = _grouped_gemm(xs, wp.reshape(P, 1), w1, b1, w3, b3, w2, b2,
                       be.reshape(NB), bv.reshape(NB))
    y = _sc_combine(eo, pos.reshape(TOPK * N))
    return _final(y, z, out_w, out_b)


def kernel(x, task_id, gate_w, W1, B1, W2, B2, W3, B3, sw1, sb1, sw2, sb2,
           sw3, sb3, out_w, out_b):
    xf = x.reshape(N, D)
    return _moe(xf, gate_w, W1, B1, W2, B2, W3, B3, sw1, sb1, sw2, sb2,
                sw3, sb3, out_w, out_b)


# speedup vs baseline: 1.9607x; 1.1937x over previous
import functools

import jax
import jax.numpy as jnp
from jax import lax
from jax.experimental import pallas as pl
from jax.experimental.pallas import tpu as pltpu
from jax.experimental.pallas import tpu_sc as plsc

E = 8
TOPK = 2
N = 2048
D = 1024
I_ = 1024
SI = 1024
OUT = 1024
ROUTE_SCALE = 1.0

BLK = 256
NB = (TOPK * N + E * (BLK - 1) + BLK - 1) // BLK
P = NB * BLK
GCH = 512
SBLK = 256
FBLK = 128

NC = 2
NS = 16
NW = NC * NS
GROWS = P // NW
GCHUNK = 32
CTOK = N // NW
CCH = 16


def _routing_body(x_ref, gw_ref, pos_ref, w_ref, be_ref, bv_ref):
    x = x_ref[...]
    gw = gw_ref[...]
    s = lax.dot_general(x, gw, (((1,), (1,)), ((), ())),
                        preferred_element_type=jnp.float32)
    m = jnp.max(s, axis=1, keepdims=True)
    p = jnp.exp(s - m)
    p = p / jnp.sum(p, axis=1, keepdims=True)

    iota_e = lax.broadcasted_iota(jnp.int32, (N, E), 1)
    v1 = jnp.max(p, axis=1, keepdims=True)
    i1 = jnp.min(jnp.where(p == v1, iota_e, E), axis=1, keepdims=True)
    pm = jnp.where(iota_e == i1, -1.0, p)
    v2 = jnp.max(pm, axis=1, keepdims=True)
    i2 = jnp.min(jnp.where(pm == v2, iota_e, E), axis=1, keepdims=True)

    oh1 = (iota_e == i1).astype(jnp.float32)
    oh2 = (iota_e == i2).astype(jnp.float32)
    oh = oh1 + oh2
    c = jnp.sum(oh, axis=0, keepdims=True)
    ssz = jnp.floor((c + (BLK - 1)) * (1.0 / BLK)) * BLK
    u8s = (lax.broadcasted_iota(jnp.int32, (E, E), 0) <
           lax.broadcasted_iota(jnp.int32, (E, E), 1)).astype(jnp.float32)
    off = lax.dot_general(ssz, u8s, (((1,), (0,)), ((), ())),
                          precision=lax.Precision.HIGHEST,
                          preferred_element_type=jnp.float32)

    bstart = lax.broadcasted_iota(jnp.int32, (1, NB), 1).astype(jnp.float32) * BLK
    be = jnp.full((1, NB), float(E - 1), jnp.float32)
    for e in range(E):
        off_e = off[0:1, e:e + 1]
        end_e = off_e + ssz[0:1, e:e + 1]
        be = jnp.where((bstart >= off_e) & (bstart < end_e), float(e), be)
    total = off[0:1, E - 1:E] + ssz[0:1, E - 1:E]
    bv = (bstart < total)
    be_ref[...] = be.astype(jnp.int32)
    bv_ref[...] = bv.astype(jnp.int32)

    lns = (lax.broadcasted_iota(jnp.int32, (N, N), 1) <
           lax.broadcasted_iota(jnp.int32, (N, N), 0)).astype(jnp.float32)
    ex = lax.dot_general(lns, oh, (((1,), (0,)), ((), ())),
                         preferred_element_type=jnp.float32)
    pos0 = jnp.sum(oh1 * (off + ex), axis=1, keepdims=True)
    pos1 = jnp.sum(oh2 * (off + ex + oh1), axis=1, keepdims=True)

    pos_ref[...] = jnp.concatenate([pos0, pos1], axis=1).astype(jnp.int32)
    w_ref[...] = jnp.concatenate([v1, v2], axis=1) * ROUTE_SCALE


def _routing(x, gate_w):
    return pl.pallas_call(
        _routing_body,
        out_shape=[
            jax.ShapeDtypeStruct((N, TOPK), jnp.int32),
            jax.ShapeDtypeStruct((N, TOPK), jnp.float32),
            jax.ShapeDtypeStruct((1, NB), jnp.int32),
            jax.ShapeDtypeStruct((1, NB), jnp.int32),
        ],
    )(x, gate_w)


TCH = 16
NCH = CTOK // TCH


def _sc_dispatch(x, p0, p1, w0, w1):
    mesh = plsc.VectorSubcoreMesh(core_axis_name="c", subcore_axis_name="s",
                                  num_cores=NC)

    @functools.partial(
        pl.kernel, mesh=mesh,
        out_type=[
            jax.ShapeDtypeStruct((P, D), jnp.float32),
            jax.ShapeDtypeStruct((P,), jnp.float32),
        ],
        scratch_types=[
            pltpu.VMEM((CTOK, D), jnp.float32),
            pltpu.VMEM((NCH, TCH), jnp.int32),
            pltpu.VMEM((NCH, TCH), jnp.int32),
            pltpu.VMEM((NCH, TCH), jnp.float32),
            pltpu.VMEM((NCH, TCH), jnp.float32),
            pltpu.SemaphoreType.DMA,
            pltpu.SemaphoreType.DMA,
            pltpu.SemaphoreType.DMA,
            pltpu.SemaphoreType.DMA,
            pltpu.SemaphoreType.DMA,
        ],
    )
    def k(x_hbm, p0_hbm, p1_hbm, w0_hbm, w1_hbm, xs_hbm, wp_hbm,
          rows_v, i0_v, i1_v, w0_v, w1_v, sem, *sem_x):
        wid = lax.axis_index("s") * NC + lax.axis_index("c")
        xh = [pltpu.async_copy(
            x_hbm.at[pl.ds(wid * CTOK + c * TCH, TCH)],
            rows_v.at[pl.ds(c * TCH, TCH)], sem_x[c]) for c in range(NCH)]
        pltpu.sync_copy(p0_hbm.at[wid], i0_v)
        pltpu.sync_copy(p1_hbm.at[wid], i1_v)
        pltpu.sync_copy(w0_hbm.at[wid], w0_v)
        pltpu.sync_copy(w1_hbm.at[wid], w1_v)
        handles = []
        for c in range(NCH):
            xh[c].wait()
            src = rows_v.at[pl.ds(c * TCH, TCH)]
            handles.append(pltpu.async_copy(src, xs_hbm.at[i0_v.at[c]], sem))
            handles.append(pltpu.async_copy(src, xs_hbm.at[i1_v.at[c]], sem))
            handles.append(pltpu.async_copy(w0_v.at[c], wp_hbm.at[i0_v.at[c]],
                                            sem))
            handles.append(pltpu.async_copy(w1_v.at[c], wp_hbm.at[i1_v.at[c]],
                                            sem))
        for h in handles:
            h.wait()

    return k(x, p0, p1, w0, w1)


def _gemm_body(be_ref, bv_ref, xs_ref, wp_ref, w1_ref, b1_ref, w3_ref,
               b3_ref, w2_ref, b2_ref, eo_ref):
    b = pl.program_id(0)

    @pl.when(bv_ref[b] == 1)
    def _():
        xs = xs_ref[...]
        h1 = lax.dot_general(xs, w1_ref[0], (((1,), (1,)), ((), ())),
                             preferred_element_type=jnp.float32) + b1_ref[0]
        h3 = lax.dot_general(xs, w3_ref[0], (((1,), (1,)), ((), ())),
                             preferred_element_type=jnp.float32) + b3_ref[0]
        h = jnp.where(h1 >= 0, h1, 0.01 * h1) * h3
        eo = lax.dot_general(h, w2_ref[0], (((1,), (1,)), ((), ())),
                             preferred_element_type=jnp.float32) + b2_ref[0]
        eo_ref[...] = eo * wp_ref[...]

    @pl.when(bv_ref[b] == 0)
    def _():
        eo_ref[...] = jnp.zeros((BLK, D), jnp.float32)


def _grouped_gemm(xs, wp, w1, b1, w3, b3, w2, b2, be, bv):
    grid_spec = pltpu.PrefetchScalarGridSpec(
        num_scalar_prefetch=2,
        grid=(NB,),
        in_specs=[
            pl.BlockSpec((BLK, D), lambda b, be, bv: (b, 0)),
            pl.BlockSpec((BLK, 1), lambda b, be, bv: (b, 0)),
            pl.BlockSpec((1, I_, D), lambda b, be, bv: (be[b], 0, 0)),
            pl.BlockSpec((1, 1, I_), lambda b, be, bv: (be[b], 0, 0)),
            pl.BlockSpec((1, I_, D), lambda b, be, bv: (be[b], 0, 0)),
            pl.BlockSpec((1, 1, I_), lambda b, be, bv: (be[b], 0, 0)),
            pl.BlockSpec((1, D, I_), lambda b, be, bv: (be[b], 0, 0)),
            pl.BlockSpec((1, 1, D), lambda b, be, bv: (be[b], 0, 0)),
        ],
        out_specs=pl.BlockSpec((BLK, D), lambda b, be, bv: (b, 0)),
    )
    return pl.pallas_call(
        _gemm_body,
        grid_spec=grid_spec,
        out_shape=jax.ShapeDtypeStruct((P, D), jnp.float32),
    )(be, bv, xs, wp, w1, b1.reshape(E, 1, I_), w3, b3.reshape(E, 1, I_),
      w2, b2.reshape(E, 1, D))


def _sc_combine(eo, pos_flat):
    mesh = plsc.VectorSubcoreMesh(core_axis_name="c", subcore_axis_name="s",
                                  num_cores=NC)

    @functools.partial(
        pl.kernel, mesh=mesh,
        out_type=jax.ShapeDtypeStruct((N, D), jnp.float32),
        scratch_types=[
            pltpu.VMEM((TOPK * CTOK,), jnp.int32),
            pltpu.VMEM((TOPK * CCH, D), jnp.float32),
            pltpu.VMEM((TOPK * CCH, D), jnp.float32),
            pltpu.VMEM((CCH, D), jnp.float32),
            pltpu.VMEM((CCH, D), jnp.float32),
            pltpu.SemaphoreType.DMA,
            pltpu.SemaphoreType.DMA,
            pltpu.SemaphoreType.DMA,
            pltpu.SemaphoreType.DMA,
        ],
    )
    def k(eo_hbm, pos_hbm, out_hbm, idx_v, rows_a, rows_b, y_a, y_b,
          sem_a, sem_b, sem_ya, sem_yb):
        wid = lax.axis_index("s") * NC + lax.axis_index("c")
        base = wid * TOPK * CTOK
        nch = CTOK // CCH
        pltpu.sync_copy(pos_hbm.at[pl.ds(base, TOPK * CTOK)], idx_v)
        bufs = [rows_a, rows_b]
        sems = [sem_a, sem_b]
        ybufs = [y_a, y_b]
        ysems = [sem_ya, sem_yb]
        handles = [None] * nch
        yhandles = [None] * nch
        handles[0] = pltpu.async_copy(
            eo_hbm.at[idx_v.at[pl.ds(0, TOPK * CCH)]], bufs[0], sems[0])
        for c in range(nch):
            handles[c].wait()
            if c + 1 < nch:
                handles[c + 1] = pltpu.async_copy(
                    eo_hbm.at[idx_v.at[pl.ds((c + 1) * TOPK * CCH,
                                             TOPK * CCH)]],
                    bufs[(c + 1) % 2], sems[(c + 1) % 2])
            rows_v = bufs[c % 2]
            y_v = ybufs[c % 2]
            if c >= 2:
                yhandles[c - 2].wait()

            def pair_add(t, carry):
                def lane_add(j, carry2):
                    y_v[t, pl.ds(j * 16, 16)] = (
                        rows_v[2 * t, pl.ds(j * 16, 16)] +
                        rows_v[2 * t + 1, pl.ds(j * 16, 16)])
                    return carry2
                return lax.fori_loop(0, D // 16, lane_add, carry)
            lax.fori_loop(0, CCH, pair_add, 0)
            yhandles[c] = pltpu.async_copy(
                y_v, out_hbm.at[pl.ds(wid * CTOK + c * CCH, CCH)],
                ysems[c % 2])
        for c in range(max(0, nch - 2), nch):
            yhandles[c].wait()

    return k(eo, pos_flat)


def _shared_body(x_ref, sw1_ref, sb1_ref, sw3_ref, sb3_ref, sw2_ref, sb2_ref,
                 z_ref):
    x = x_ref[...]
    s1 = lax.dot_general(x, sw1_ref[...], (((1,), (1,)), ((), ())),
                         preferred_element_type=jnp.float32) + sb1_ref[...]
    s3 = lax.dot_general(x, sw3_ref[...], (((1,), (1,)), ((), ())),
                         preferred_element_type=jnp.float32) + sb3_ref[...]
    h = jnp.where(s1 >= 0, s1, 0.01 * s1) * s3
    z_ref[...] = lax.dot_general(h, sw2_ref[...], (((1,), (1,)), ((), ())),
                                 preferred_element_type=jnp.float32) + sb2_ref[...]


def _shared_ffn(x, sw1, sb1, sw3, sb3, sw2, sb2):
    return pl.pallas_call(
        _shared_body,
        grid=(N // SBLK,),
        in_specs=[
            pl.BlockSpec((SBLK, D), lambda b: (b, 0)),
            pl.BlockSpec((SI, D), lambda b: (0, 0)),
            pl.BlockSpec((1, SI), lambda b: (0, 0)),
            pl.BlockSpec((SI, D), lambda b: (0, 0)),
            pl.BlockSpec((1, SI), lambda b: (0, 0)),
            pl.BlockSpec((D, SI), lambda b: (0, 0)),
            pl.BlockSpec((1, D), lambda b: (0, 0)),
        ],
        out_specs=pl.BlockSpec((SBLK, D), lambda b: (b, 0)),
        out_shape=jax.ShapeDtypeStruct((N, D), jnp.float32),
    )(x, sw1, sb1.reshape(1, SI), sw3, sb3.reshape(1, SI), sw2,
      sb2.reshape(1, D))


def _final_body(y_ref, z_ref, ow_ref, ob_ref, out_ref):
    yz = y_ref[...] + z_ref[...]
    out_ref[...] = lax.dot_general(yz, ow_ref[...], (((1,), (1,)), ((), ())),
                                   preferred_element_type=jnp.float32) + ob_ref[...]


def _final(y, z, out_w, out_b):
    return pl.pallas_call(
        _final_body,
        grid=(N // FBLK,),
        in_specs=[
            pl.BlockSpec((FBLK, D), lambda b: (b, 0)),
            pl.BlockSpec((FBLK, D), lambda b: (b, 0)),
            pl.BlockSpec((OUT, D), lambda b: (0, 0)),
            pl.BlockSpec((1, OUT), lambda b: (0, 0)),
        ],
        out_specs=pl.BlockSpec((FBLK, OUT), lambda b: (b, 0)),
        out_shape=jax.ShapeDtypeStruct((N, OUT), jnp.float32),
    )(y, z, out_w, out_b.reshape(1, OUT))


@jax.jit
def _moe(x, gate_w, w1, b1, w2, b2, w3, b3, sw1, sb1, sw2, sb2, sw3, sb3,
         out_w, out_b):
    pos, wts, be, bv = _routing(x, gate_w)
    xs, wp = _sc_dispatch(
        x,
        pos[:, 0].reshape(NW, NCH, TCH), pos[:, 1].reshape(NW, NCH, TCH),
        wts[:, 0].reshape(NW, NCH, TCH), wts[:, 1].reshape(NW, NCH, TCH))
    z = _shared_ffn(x, sw1, sb1, sw3, sb3, sw2, sb2)
    eo = _grouped_gemm(xs, wp.reshape(P, 1), w1, b1, w3, b3, w2, b2,
                       be.reshape(NB), bv.reshape(NB))
    y = _sc_combine(eo, pos.reshape(TOPK * N))
    return _final(y, z, out_w, out_b)


def kernel(x, task_id, gate_w, W1, B1, W2, B2, W3, B3, sw1, sb1, sw2, sb2,
           sw3, sb3, out_w, out_b):
    xf = x.reshape(N, D)
    return _moe(xf, gate_w, W1, B1, W2, B2, W3, B3, sw1, sb1, sw2, sb2,
                sw3, sb3, out_w, out_b)


# speedup vs baseline: 2.0523x; 1.0467x over previous
import functools

import jax
import jax.numpy as jnp
from jax import lax
from jax.experimental import pallas as pl
from jax.experimental.pallas import tpu as pltpu
from jax.experimental.pallas import tpu_sc as plsc

E = 8
TOPK = 2
N = 2048
D = 1024
I_ = 1024
SI = 1024
OUT = 1024
ROUTE_SCALE = 1.0

BLK = 512
NB = (TOPK * N + E * (BLK - 1) + BLK - 1) // BLK
P = NB * BLK
GCH = 512
SBLK = 256
FBLK = 128

NC = 2
NS = 16
NW = NC * NS
GROWS = P // NW
GCHUNK = 32
CTOK = N // NW
CCH = 16


def _routing_body(x_ref, gw_ref, pos_ref, w_ref, be_ref, bv_ref):
    x = x_ref[...]
    gw = gw_ref[...]
    s = lax.dot_general(x, gw, (((1,), (1,)), ((), ())),
                        preferred_element_type=jnp.float32)
    m = jnp.max(s, axis=1, keepdims=True)
    p = jnp.exp(s - m)
    p = p / jnp.sum(p, axis=1, keepdims=True)

    iota_e = lax.broadcasted_iota(jnp.int32, (N, E), 1)
    v1 = jnp.max(p, axis=1, keepdims=True)
    i1 = jnp.min(jnp.where(p == v1, iota_e, E), axis=1, keepdims=True)
    pm = jnp.where(iota_e == i1, -1.0, p)
    v2 = jnp.max(pm, axis=1, keepdims=True)
    i2 = jnp.min(jnp.where(pm == v2, iota_e, E), axis=1, keepdims=True)

    oh1 = (iota_e == i1).astype(jnp.float32)
    oh2 = (iota_e == i2).astype(jnp.float32)
    oh = oh1 + oh2
    c = jnp.sum(oh, axis=0, keepdims=True)
    ssz = jnp.floor((c + (BLK - 1)) * (1.0 / BLK)) * BLK
    u8s = (lax.broadcasted_iota(jnp.int32, (E, E), 0) <
           lax.broadcasted_iota(jnp.int32, (E, E), 1)).astype(jnp.float32)
    off = lax.dot_general(ssz, u8s, (((1,), (0,)), ((), ())),
                          precision=lax.Precision.HIGHEST,
                          preferred_element_type=jnp.float32)

    bstart = lax.broadcasted_iota(jnp.int32, (1, NB), 1).astype(jnp.float32) * BLK
    be = jnp.full((1, NB), float(E - 1), jnp.float32)
    for e in range(E):
        off_e = off[0:1, e:e + 1]
        end_e = off_e + ssz[0:1, e:e + 1]
        be = jnp.where((bstart >= off_e) & (bstart < end_e), float(e), be)
    total = off[0:1, E - 1:E] + ssz[0:1, E - 1:E]
    bv = (bstart < total)
    be_ref[...] = be.astype(jnp.int32)
    bv_ref[...] = bv.astype(jnp.int32)

    lns = (lax.broadcasted_iota(jnp.int32, (N, N), 1) <
           lax.broadcasted_iota(jnp.int32, (N, N), 0)).astype(jnp.float32)
    ex = lax.dot_general(lns, oh, (((1,), (0,)), ((), ())),
                         preferred_element_type=jnp.float32)
    pos0 = jnp.sum(oh1 * (off + ex), axis=1, keepdims=True)
    pos1 = jnp.sum(oh2 * (off + ex + oh1), axis=1, keepdims=True)

    pos_ref[...] = jnp.concatenate([pos0, pos1], axis=1).astype(jnp.int32)
    w_ref[...] = jnp.concatenate([v1, v2], axis=1) * ROUTE_SCALE


def _routing(x, gate_w):
    return pl.pallas_call(
        _routing_body,
        out_shape=[
            jax.ShapeDtypeStruct((N, TOPK), jnp.int32),
            jax.ShapeDtypeStruct((N, TOPK), jnp.float32),
            jax.ShapeDtypeStruct((1, NB), jnp.int32),
            jax.ShapeDtypeStruct((1, NB), jnp.int32),
        ],
    )(x, gate_w)


TCH = 16
NCH = CTOK // TCH


def _sc_dispatch(x, p0, p1, w0, w1):
    mesh = plsc.VectorSubcoreMesh(core_axis_name="c", subcore_axis_name="s",
                                  num_cores=NC)

    @functools.partial(
        pl.kernel, mesh=mesh,
        out_type=[
            jax.ShapeDtypeStruct((P, D), jnp.float32),
            jax.ShapeDtypeStruct((P,), jnp.float32),
        ],
        scratch_types=[
            pltpu.VMEM((CTOK, D), jnp.float32),
            pltpu.VMEM((NCH, TCH), jnp.int32),
            pltpu.VMEM((NCH, TCH), jnp.int32),
            pltpu.VMEM((NCH, TCH), jnp.float32),
            pltpu.VMEM((NCH, TCH), jnp.float32),
            pltpu.SemaphoreType.DMA,
            pltpu.SemaphoreType.DMA,
            pltpu.SemaphoreType.DMA,
            pltpu.SemaphoreType.DMA,
            pltpu.SemaphoreType.DMA,
        ],
    )
    def k(x_hbm, p0_hbm, p1_hbm, w0_hbm, w1_hbm, xs_hbm, wp_hbm,
          rows_v, i0_v, i1_v, w0_v, w1_v, sem, *sem_x):
        wid = lax.axis_index("s") * NC + lax.axis_index("c")
        xh = [pltpu.async_copy(
            x_hbm.at[pl.ds(wid * CTOK + c * TCH, TCH)],
            rows_v.at[pl.ds(c * TCH, TCH)], sem_x[c]) for c in range(NCH)]
        pltpu.sync_copy(p0_hbm.at[wid], i0_v)
        pltpu.sync_copy(p1_hbm.at[wid], i1_v)
        pltpu.sync_copy(w0_hbm.at[wid], w0_v)
        pltpu.sync_copy(w1_hbm.at[wid], w1_v)
        handles = []
        for c in range(NCH):
            xh[c].wait()
            src = rows_v.at[pl.ds(c * TCH, TCH)]
            handles.append(pltpu.async_copy(src, xs_hbm.at[i0_v.at[c]], sem))
            handles.append(pltpu.async_copy(src, xs_hbm.at[i1_v.at[c]], sem))
            handles.append(pltpu.async_copy(w0_v.at[c], wp_hbm.at[i0_v.at[c]],
                                            sem))
            handles.append(pltpu.async_copy(w1_v.at[c], wp_hbm.at[i1_v.at[c]],
                                            sem))
        for h in handles:
            h.wait()

    return k(x, p0, p1, w0, w1)


def _gemm_body(be_ref, bv_ref, xs_ref, wp_ref, w1_ref, b1_ref, w3_ref,
               b3_ref, w2_ref, b2_ref, eo_ref):
    b = pl.program_id(0)

    @pl.when(bv_ref[b] == 1)
    def _():
        xs = xs_ref[...]
        h1 = lax.dot_general(xs, w1_ref[0], (((1,), (1,)), ((), ())),
                             preferred_element_type=jnp.float32) + b1_ref[0]
        h3 = lax.dot_general(xs, w3_ref[0], (((1,), (1,)), ((), ())),
                             preferred_element_type=jnp.float32) + b3_ref[0]
        h = jnp.where(h1 >= 0, h1, 0.01 * h1) * h3
        eo = lax.dot_general(h, w2_ref[0], (((1,), (1,)), ((), ())),
                             preferred_element_type=jnp.float32) + b2_ref[0]
        eo_ref[...] = eo * wp_ref[...]

    @pl.when(bv_ref[b] == 0)
    def _():
        eo_ref[...] = jnp.zeros((BLK, D), jnp.float32)


def _grouped_gemm(xs, wp, w1, b1, w3, b3, w2, b2, be, bv):
    grid_spec = pltpu.PrefetchScalarGridSpec(
        num_scalar_prefetch=2,
        grid=(NB,),
        in_specs=[
            pl.BlockSpec((BLK, D), lambda b, be, bv: (b, 0)),
            pl.BlockSpec((BLK, 1), lambda b, be, bv: (b, 0)),
            pl.BlockSpec((1, I_, D), lambda b, be, bv: (be[b], 0, 0)),
            pl.BlockSpec((1, 1, I_), lambda b, be, bv: (be[b], 0, 0)),
            pl.BlockSpec((1, I_, D), lambda b, be, bv: (be[b], 0, 0)),
            pl.BlockSpec((1, 1, I_), lambda b, be, bv: (be[b], 0, 0)),
            pl.BlockSpec((1, D, I_), lambda b, be, bv: (be[b], 0, 0)),
            pl.BlockSpec((1, 1, D), lambda b, be, bv: (be[b], 0, 0)),
        ],
        out_specs=pl.BlockSpec((BLK, D), lambda b, be, bv: (b, 0)),
    )
    return pl.pallas_call(
        _gemm_body,
        grid_spec=grid_spec,
        out_shape=jax.ShapeDtypeStruct((P, D), jnp.float32),
    )(be, bv, xs, wp, w1, b1.reshape(E, 1, I_), w3, b3.reshape(E, 1, I_),
      w2, b2.reshape(E, 1, D))


def _sc_combine(eo, pos_flat):
    mesh = plsc.VectorSubcoreMesh(core_axis_name="c", subcore_axis_name="s",
                                  num_cores=NC)

    @functools.partial(
        pl.kernel, mesh=mesh,
        out_type=jax.ShapeDtypeStruct((N, D), jnp.float32),
        scratch_types=[
            pltpu.VMEM((TOPK * CTOK,), jnp.int32),
            pltpu.VMEM((TOPK * CCH, D), jnp.float32),
            pltpu.VMEM((TOPK * CCH, D), jnp.float32),
            pltpu.VMEM((CCH, D), jnp.float32),
            pltpu.VMEM((CCH, D), jnp.float32),
            pltpu.SemaphoreType.DMA,
            pltpu.SemaphoreType.DMA,
            pltpu.SemaphoreType.DMA,
            pltpu.SemaphoreType.DMA,
        ],
    )
    def k(eo_hbm, pos_hbm, out_hbm, idx_v, rows_a, rows_b, y_a, y_b,
          sem_a, sem_b, sem_ya, sem_yb):
        wid = lax.axis_index("s") * NC + lax.axis_index("c")
        base = wid * TOPK * CTOK
        nch = CTOK // CCH
        pltpu.sync_copy(pos_hbm.at[pl.ds(base, TOPK * CTOK)], idx_v)
        bufs = [rows_a, rows_b]
        sems = [sem_a, sem_b]
        ybufs = [y_a, y_b]
        ysems = [sem_ya, sem_yb]
        handles = [None] * nch
        yhandles = [None] * nch
        handles[0] = pltpu.async_copy(
            eo_hbm.at[idx_v.at[pl.ds(0, TOPK * CCH)]], bufs[0], sems[0])
        for c in range(nch):
            handles[c].wait()
            if c + 1 < nch:
                handles[c + 1] = pltpu.async_copy(
                    eo_hbm.at[idx_v.at[pl.ds((c + 1) * TOPK * CCH,
                                             TOPK * CCH)]],
                    bufs[(c + 1) % 2], sems[(c + 1) % 2])
            rows_v = bufs[c % 2]
            y_v = ybufs[c % 2]
            if c >= 2:
                yhandles[c - 2].wait()

            def pair_add(t, carry):
                def lane_add(j, carry2):
                    y_v[t, pl.ds(j * 16, 16)] = (
                        rows_v[2 * t, pl.ds(j * 16, 16)] +
                        rows_v[2 * t + 1, pl.ds(j * 16, 16)])
                    return carry2
                return lax.fori_loop(0, D // 16, lane_add, carry)
            lax.fori_loop(0, CCH, pair_add, 0)
            yhandles[c] = pltpu.async_copy(
                y_v, out_hbm.at[pl.ds(wid * CTOK + c * CCH, CCH)],
                ysems[c % 2])
        for c in range(max(0, nch - 2), nch):
            yhandles[c].wait()

    return k(eo, pos_flat)


def _shared_body(x_ref, sw1_ref, sb1_ref, sw3_ref, sb3_ref, sw2_ref, sb2_ref,
                 z_ref):
    x = x_ref[...]
    s1 = lax.dot_general(x, sw1_ref[...], (((1,), (1,)), ((), ())),
                         preferred_element_type=jnp.float32) + sb1_ref[...]
    s3 = lax.dot_general(x, sw3_ref[...], (((1,), (1,)), ((), ())),
                         preferred_element_type=jnp.float32) + sb3_ref[...]
    h = jnp.where(s1 >= 0, s1, 0.01 * s1) * s3
    z_ref[...] = lax.dot_general(h, sw2_ref[...], (((1,), (1,)), ((), ())),
                                 preferred_element_type=jnp.float32) + sb2_ref[...]


def _shared_ffn(x, sw1, sb1, sw3, sb3, sw2, sb2):
    return pl.pallas_call(
        _shared_body,
        grid=(N // SBLK,),
        in_specs=[
            pl.BlockSpec((SBLK, D), lambda b: (b, 0)),
            pl.BlockSpec((SI, D), lambda b: (0, 0)),
            pl.BlockSpec((1, SI), lambda b: (0, 0)),
            pl.BlockSpec((SI, D), lambda b: (0, 0)),
            pl.BlockSpec((1, SI), lambda b: (0, 0)),
            pl.BlockSpec((D, SI), lambda b: (0, 0)),
            pl.BlockSpec((1, D), lambda b: (0, 0)),
        ],
        out_specs=pl.BlockSpec((SBLK, D), lambda b: (b, 0)),
        out_shape=jax.ShapeDtypeStruct((N, D), jnp.float32),
    )(x, sw1, sb1.reshape(1, SI), sw3, sb3.reshape(1, SI), sw2,
      sb2.reshape(1, D))


def _final_body(y_ref, z_ref, ow_ref, ob_ref, out_ref):
    yz = y_ref[...] + z_ref[...]
    out_ref[...] = lax.dot_general(yz, ow_ref[...], (((1,), (1,)), ((), ())),
                                   preferred_element_type=jnp.float32) + ob_ref[...]


def _final(y, z, out_w, out_b):
    return pl.pallas_call(
        _final_body,
        grid=(N // FBLK,),
        in_specs=[
            pl.BlockSpec((FBLK, D), lambda b: (b, 0)),
            pl.BlockSpec((FBLK, D), lambda b: (b, 0)),
            pl.BlockSpec((OUT, D), lambda b: (0, 0)),
            pl.BlockSpec((1, OUT), lambda b: (0, 0)),
        ],
        out_specs=pl.BlockSpec((FBLK, OUT), lambda b: (b, 0)),
        out_shape=jax.ShapeDtypeStruct((N, OUT), jnp.float32),
    )(y, z, out_w, out_b.reshape(1, OUT))


@jax.jit
def _moe(x, gate_w, w1, b1, w2, b2, w3, b3, sw1, sb1, sw2, sb2, sw3, sb3,
         out_w, out_b):
    pos, wts, be, bv = _routing(x, gate_w)
    xs, wp = _sc_dispatch(
        x,
        pos[:, 0].reshape(NW, NCH, TCH), pos[:, 1].reshape(NW, NCH, TCH),
        wts[:, 0].reshape(NW, NCH, TCH), wts[:, 1].reshape(NW, NCH, TCH))
    z = _shared_ffn(x, sw1, sb1, sw3, sb3, sw2, sb2)
    eo = _grouped_gemm(xs, wp.reshape(P, 1), w1, b1, w3, b3, w2, b2,
                       be.reshape(NB), bv.reshape(NB))
    y = _sc_combine(eo, pos.reshape(TOPK * N))
    return _final(y, z, out_w, out_b)


def kernel(x, task_id, gate_w, W1, B1, W2, B2, W3, B3, sw1, sb1, sw2, sb2,
           sw3, sb3, out_w, out_b):
    xf = x.reshape(N, D)
    return _moe(xf, gate_w, W1, B1, W2, B2, W3, B3, sw1, sb1, sw2, sb2,
                sw3, sb3, out_w, out_b)


# speedup vs baseline: 2.1358x; 1.0407x over previous
import functools

import jax
import jax.numpy as jnp
from jax import lax
from jax.experimental import pallas as pl
from jax.experimental.pallas import tpu as pltpu
from jax.experimental.pallas import tpu_sc as plsc

E = 8
TOPK = 2
N = 2048
D = 1024
I_ = 1024
SI = 1024
OUT = 1024
ROUTE_SCALE = 1.0

BLK = 512
NB = (TOPK * N + E * (BLK - 1) + BLK - 1) // BLK
P = NB * BLK
GCH = 512
SBLK = 512
FBLK = 256

NC = 2
NS = 16
NW = NC * NS
GROWS = P // NW
GCHUNK = 32
CTOK = N // NW
CCH = 16


def _routing_body(x_ref, gw_ref, pos_ref, w_ref, be_ref, bv_ref):
    x = x_ref[...]
    gw = gw_ref[...]
    s = lax.dot_general(x, gw, (((1,), (1,)), ((), ())),
                        preferred_element_type=jnp.float32)
    m = jnp.max(s, axis=1, keepdims=True)
    p = jnp.exp(s - m)
    p = p / jnp.sum(p, axis=1, keepdims=True)

    iota_e = lax.broadcasted_iota(jnp.int32, (N, E), 1)
    v1 = jnp.max(p, axis=1, keepdims=True)
    i1 = jnp.min(jnp.where(p == v1, iota_e, E), axis=1, keepdims=True)
    pm = jnp.where(iota_e == i1, -1.0, p)
    v2 = jnp.max(pm, axis=1, keepdims=True)
    i2 = jnp.min(jnp.where(pm == v2, iota_e, E), axis=1, keepdims=True)

    oh1 = (iota_e == i1).astype(jnp.float32)
    oh2 = (iota_e == i2).astype(jnp.float32)
    oh = oh1 + oh2
    c = jnp.sum(oh, axis=0, keepdims=True)
    ssz = jnp.floor((c + (BLK - 1)) * (1.0 / BLK)) * BLK
    u8s = (lax.broadcasted_iota(jnp.int32, (E, E), 0) <
           lax.broadcasted_iota(jnp.int32, (E, E), 1)).astype(jnp.float32)
    off = lax.dot_general(ssz, u8s, (((1,), (0,)), ((), ())),
                          precision=lax.Precision.HIGHEST,
                          preferred_element_type=jnp.float32)

    bstart = lax.broadcasted_iota(jnp.int32, (1, NB), 1).astype(jnp.float32) * BLK
    be = jnp.full((1, NB), float(E - 1), jnp.float32)
    for e in range(E):
        off_e = off[0:1, e:e + 1]
        end_e = off_e + ssz[0:1, e:e + 1]
        be = jnp.where((bstart >= off_e) & (bstart < end_e), float(e), be)
    total = off[0:1, E - 1:E] + ssz[0:1, E - 1:E]
    bv = (bstart < total)
    be_ref[...] = be.astype(jnp.int32)
    bv_ref[...] = bv.astype(jnp.int32)

    lns = (lax.broadcasted_iota(jnp.int32, (N, N), 1) <
           lax.broadcasted_iota(jnp.int32, (N, N), 0)).astype(jnp.float32)
    ex = lax.dot_general(lns, oh, (((1,), (0,)), ((), ())),
                         preferred_element_type=jnp.float32)
    pos0 = jnp.sum(oh1 * (off + ex), axis=1, keepdims=True)
    pos1 = jnp.sum(oh2 * (off + ex + oh1), axis=1, keepdims=True)

    pos_ref[...] = jnp.concatenate([pos0, pos1], axis=1).astype(jnp.int32)
    w_ref[...] = jnp.concatenate([v1, v2], axis=1) * ROUTE_SCALE


def _routing(x, gate_w):
    return pl.pallas_call(
        _routing_body,
        out_shape=[
            jax.ShapeDtypeStruct((N, TOPK), jnp.int32),
            jax.ShapeDtypeStruct((N, TOPK), jnp.float32),
            jax.ShapeDtypeStruct((1, NB), jnp.int32),
            jax.ShapeDtypeStruct((1, NB), jnp.int32),
        ],
    )(x, gate_w)


TCH = 16
NCH = CTOK // TCH


def _sc_dispatch(x, p0, p1, w0, w1):
    mesh = plsc.VectorSubcoreMesh(core_axis_name="c", subcore_axis_name="s",
                                  num_cores=NC)

    @functools.partial(
        pl.kernel, mesh=mesh,
        out_type=[
            jax.ShapeDtypeStruct((P, D), jnp.float32),
            jax.ShapeDtypeStruct((P,), jnp.float32),
        ],
        scratch_types=[
            pltpu.VMEM((CTOK, D), jnp.float32),
            pltpu.VMEM((NCH, TCH), jnp.int32),
            pltpu.VMEM((NCH, TCH), jnp.int32),
            pltpu.VMEM((NCH, TCH), jnp.float32),
            pltpu.VMEM((NCH, TCH), jnp.float32),
            pltpu.SemaphoreType.DMA,
            pltpu.SemaphoreType.DMA,
            pltpu.SemaphoreType.DMA,
            pltpu.SemaphoreType.DMA,
            pltpu.SemaphoreType.DMA,
        ],
    )
    def k(x_hbm, p0_hbm, p1_hbm, w0_hbm, w1_hbm, xs_hbm, wp_hbm,
          rows_v, i0_v, i1_v, w0_v, w1_v, sem, *sem_x):
        wid = lax.axis_index("s") * NC + lax.axis_index("c")
        xh = [pltpu.async_copy(
            x_hbm.at[pl.ds(wid * CTOK + c * TCH, TCH)],
            rows_v.at[pl.ds(c * TCH, TCH)], sem_x[c]) for c in range(NCH)]
        pltpu.sync_copy(p0_hbm.at[wid], i0_v)
        pltpu.sync_copy(p1_hbm.at[wid], i1_v)
        pltpu.sync_copy(w0_hbm.at[wid], w0_v)
        pltpu.sync_copy(w1_hbm.at[wid], w1_v)
        handles = []
        for c in range(NCH):
            xh[c].wait()
            src = rows_v.at[pl.ds(c * TCH, TCH)]
            handles.append(pltpu.async_copy(src, xs_hbm.at[i0_v.at[c]], sem))
            handles.append(pltpu.async_copy(src, xs_hbm.at[i1_v.at[c]], sem))
            handles.append(pltpu.async_copy(w0_v.at[c], wp_hbm.at[i0_v.at[c]],
                                            sem))
            handles.append(pltpu.async_copy(w1_v.at[c], wp_hbm.at[i1_v.at[c]],
                                            sem))
        for h in handles:
            h.wait()

    return k(x, p0, p1, w0, w1)


def _gemm_body(be_ref, bv_ref, xs_ref, wp_ref, w1_ref, b1_ref, w3_ref,
               b3_ref, w2_ref, b2_ref, eo_ref):
    b = pl.program_id(0)

    @pl.when(bv_ref[b] == 1)
    def _():
        xs = xs_ref[...]
        h1 = lax.dot_general(xs, w1_ref[0], (((1,), (1,)), ((), ())),
                             preferred_element_type=jnp.float32) + b1_ref[0]
        h3 = lax.dot_general(xs, w3_ref[0], (((1,), (1,)), ((), ())),
                             preferred_element_type=jnp.float32) + b3_ref[0]
        h = jnp.where(h1 >= 0, h1, 0.01 * h1) * h3
        eo = lax.dot_general(h, w2_ref[0], (((1,), (1,)), ((), ())),
                             preferred_element_type=jnp.float32) + b2_ref[0]
        eo_ref[...] = eo * wp_ref[...]

    @pl.when(bv_ref[b] == 0)
    def _():
        eo_ref[...] = jnp.zeros((BLK, D), jnp.float32)


def _grouped_gemm(xs, wp, w1, b1, w3, b3, w2, b2, be, bv):
    grid_spec = pltpu.PrefetchScalarGridSpec(
        num_scalar_prefetch=2,
        grid=(NB,),
        in_specs=[
            pl.BlockSpec((BLK, D), lambda b, be, bv: (b, 0)),
            pl.BlockSpec((BLK, 1), lambda b, be, bv: (b, 0)),
            pl.BlockSpec((1, I_, D), lambda b, be, bv: (be[b], 0, 0)),
            pl.BlockSpec((1, 1, I_), lambda b, be, bv: (be[b], 0, 0)),
            pl.BlockSpec((1, I_, D), lambda b, be, bv: (be[b], 0, 0)),
            pl.BlockSpec((1, 1, I_), lambda b, be, bv: (be[b], 0, 0)),
            pl.BlockSpec((1, D, I_), lambda b, be, bv: (be[b], 0, 0)),
            pl.BlockSpec((1, 1, D), lambda b, be, bv: (be[b], 0, 0)),
        ],
        out_specs=pl.BlockSpec((BLK, D), lambda b, be, bv: (b, 0)),
    )
    return pl.pallas_call(
        _gemm_body,
        grid_spec=grid_spec,
        out_shape=jax.ShapeDtypeStruct((P, D), jnp.float32),
    )(be, bv, xs, wp, w1, b1.reshape(E, 1, I_), w3, b3.reshape(E, 1, I_),
      w2, b2.reshape(E, 1, D))


def _sc_combine(eo, pos_flat):
    mesh = plsc.VectorSubcoreMesh(core_axis_name="c", subcore_axis_name="s",
                                  num_cores=NC)

    @functools.partial(
        pl.kernel, mesh=mesh,
        out_type=jax.ShapeDtypeStruct((N, D), jnp.float32),
        scratch_types=[
            pltpu.VMEM((TOPK * CTOK,), jnp.int32),
            pltpu.VMEM((TOPK * CCH, D), jnp.float32),
            pltpu.VMEM((TOPK * CCH, D), jnp.float32),
            pltpu.VMEM((CCH, D), jnp.float32),
            pltpu.VMEM((CCH, D), jnp.float32),
            pltpu.SemaphoreType.DMA,
            pltpu.SemaphoreType.DMA,
            pltpu.SemaphoreType.DMA,
            pltpu.SemaphoreType.DMA,
        ],
    )
    def k(eo_hbm, pos_hbm, out_hbm, idx_v, rows_a, rows_b, y_a, y_b,
          sem_a, sem_b, sem_ya, sem_yb):
        wid = lax.axis_index("s") * NC + lax.axis_index("c")
        base = wid * TOPK * CTOK
        nch = CTOK // CCH
        pltpu.sync_copy(pos_hbm.at[pl.ds(base, TOPK * CTOK)], idx_v)
        bufs = [rows_a, rows_b]
        sems = [sem_a, sem_b]
        ybufs = [y_a, y_b]
        ysems = [sem_ya, sem_yb]
        handles = [None] * nch
        yhandles = [None] * nch
        handles[0] = pltpu.async_copy(
            eo_hbm.at[idx_v.at[pl.ds(0, TOPK * CCH)]], bufs[0], sems[0])
        for c in range(nch):
            handles[c].wait()
            if c + 1 < nch:
                handles[c + 1] = pltpu.async_copy(
                    eo_hbm.at[idx_v.at[pl.ds((c + 1) * TOPK * CCH,
                                             TOPK * CCH)]],
                    bufs[(c + 1) % 2], sems[(c + 1) % 2])
            rows_v = bufs[c % 2]
            y_v = ybufs[c % 2]
            if c >= 2:
                yhandles[c - 2].wait()

            def pair_add(t, carry):
                def lane_add(j, carry2):
                    y_v[t, pl.ds(j * 16, 16)] = (
                        rows_v[2 * t, pl.ds(j * 16, 16)] +
                        rows_v[2 * t + 1, pl.ds(j * 16, 16)])
                    return carry2
                return lax.fori_loop(0, D // 16, lane_add, carry)
            lax.fori_loop(0, CCH, pair_add, 0)
            yhandles[c] = pltpu.async_copy(
                y_v, out_hbm.at[pl.ds(wid * CTOK + c * CCH, CCH)],
                ysems[c % 2])
        for c in range(max(0, nch - 2), nch):
            yhandles[c].wait()

    return k(eo, pos_flat)


def _shared_body(x_ref, sw1_ref, sb1_ref, sw3_ref, sb3_ref, sw2_ref, sb2_ref,
                 z_ref):
    x = x_ref[...]
    s1 = lax.dot_general(x, sw1_ref[...], (((1,), (1,)), ((), ())),
                         preferred_element_type=jnp.float32) + sb1_ref[...]
    s3 = lax.dot_general(x, sw3_ref[...], (((1,), (1,)), ((), ())),
                         preferred_element_type=jnp.float32) + sb3_ref[...]
    h = jnp.where(s1 >= 0, s1, 0.01 * s1) * s3
    z_ref[...] = lax.dot_general(h, sw2_ref[...], (((1,), (1,)), ((), ())),
                                 preferred_element_type=jnp.float32) + sb2_ref[...]


def _shared_ffn(x, sw1, sb1, sw3, sb3, sw2, sb2):
    return pl.pallas_call(
        _shared_body,
        grid=(N // SBLK,),
        in_specs=[
            pl.BlockSpec((SBLK, D), lambda b: (b, 0)),
            pl.BlockSpec((SI, D), lambda b: (0, 0)),
            pl.BlockSpec((1, SI), lambda b: (0, 0)),
            pl.BlockSpec((SI, D), lambda b: (0, 0)),
            pl.BlockSpec((1, SI), lambda b: (0, 0)),
            pl.BlockSpec((D, SI), lambda b: (0, 0)),
            pl.BlockSpec((1, D), lambda b: (0, 0)),
        ],
        out_specs=pl.BlockSpec((SBLK, D), lambda b: (b, 0)),
        out_shape=jax.ShapeDtypeStruct((N, D), jnp.float32),
    )(x, sw1, sb1.reshape(1, SI), sw3, sb3.reshape(1, SI), sw2,
      sb2.reshape(1, D))


def _final_body(y_ref, z_ref, ow_ref, ob_ref, out_ref):
    yz = y_ref[...] + z_ref[...]
    out_ref[...] = lax.dot_general(yz, ow_ref[...], (((1,), (1,)), ((), ())),
                                   preferred_element_type=jnp.float32) + ob_ref[...]


def _final(y, z, out_w, out_b):
    return pl.pallas_call(
        _final_body,
        grid=(N // FBLK,),
        in_specs=[
            pl.BlockSpec((FBLK, D), lambda b: (b, 0)),
            pl.BlockSpec((FBLK, D), lambda b: (b, 0)),
            pl.BlockSpec((OUT, D), lambda b: (0, 0)),
            pl.BlockSpec((1, OUT), lambda b: (0, 0)),
        ],
        out_specs=pl.BlockSpec((FBLK, OUT), lambda b: (b, 0)),
        out_shape=jax.ShapeDtypeStruct((N, OUT), jnp.float32),
    )(y, z, out_w, out_b.reshape(1, OUT))


@jax.jit
def _moe(x, gate_w, w1, b1, w2, b2, w3, b3, sw1, sb1, sw2, sb2, sw3, sb3,
         out_w, out_b):
    pos, wts, be, bv = _routing(x, gate_w)
    xs, wp = _sc_dispatch(
        x,
        pos[:, 0].reshape(NW, NCH, TCH), pos[:, 1].reshape(NW, NCH, TCH),
        wts[:, 0].reshape(NW, NCH, TCH), wts[:, 1].reshape(NW, NCH, TCH))
    z = _shared_ffn(x, sw1, sb1, sw3, sb3, sw2, sb2)
    eo = _grouped_gemm(xs, wp.reshape(P, 1), w1, b1, w3, b3, w2, b2,
                       be.reshape(NB), bv.reshape(NB))
    y = _sc_combine(eo, pos.reshape(TOPK * N))
    return _final(y, z, out_w, out_b)


def kernel(x, task_id, gate_w, W1, B1, W2, B2, W3, B3, sw1, sb1, sw2, sb2,
           sw3, sb3, out_w, out_b):
    xf = x.reshape(N, D)
    return _moe(xf, gate_w, W1, B1, W2, B2, W3, B3, sw1, sb1, sw2, sb2,
                sw3, sb3, out_w, out_b)


# speedup vs baseline: 2.1597x; 1.0112x over previous
import functools

import jax
import jax.numpy as jnp
from jax import lax
from jax.experimental import pallas as pl
from jax.experimental.pallas import tpu as pltpu
from jax.experimental.pallas import tpu_sc as plsc

E = 8
TOPK = 2
N = 2048
D = 1024
I_ = 1024
SI = 1024
OUT = 1024
ROUTE_SCALE = 1.0

BLK = 512
NB = (TOPK * N + E * (BLK - 1) + BLK - 1) // BLK
P = NB * BLK
GCH = 512
SBLK = 1024
FBLK = 512

NC = 2
NS = 16
NW = NC * NS
GROWS = P // NW
GCHUNK = 32
CTOK = N // NW
CCH = 16


def _routing_body(x_ref, gw_ref, pos_ref, w_ref, be_ref, bv_ref):
    x = x_ref[...]
    gw = gw_ref[...]
    s = lax.dot_general(x, gw, (((1,), (1,)), ((), ())),
                        preferred_element_type=jnp.float32)
    m = jnp.max(s, axis=1, keepdims=True)
    p = jnp.exp(s - m)
    p = p / jnp.sum(p, axis=1, keepdims=True)

    iota_e = lax.broadcasted_iota(jnp.int32, (N, E), 1)
    v1 = jnp.max(p, axis=1, keepdims=True)
    i1 = jnp.min(jnp.where(p == v1, iota_e, E), axis=1, keepdims=True)
    pm = jnp.where(iota_e == i1, -1.0, p)
    v2 = jnp.max(pm, axis=1, keepdims=True)
    i2 = jnp.min(jnp.where(pm == v2, iota_e, E), axis=1, keepdims=True)

    oh1 = (iota_e == i1).astype(jnp.float32)
    oh2 = (iota_e == i2).astype(jnp.float32)
    oh = oh1 + oh2
    c = jnp.sum(oh, axis=0, keepdims=True)
    ssz = jnp.floor((c + (BLK - 1)) * (1.0 / BLK)) * BLK
    u8s = (lax.broadcasted_iota(jnp.int32, (E, E), 0) <
           lax.broadcasted_iota(jnp.int32, (E, E), 1)).astype(jnp.float32)
    off = lax.dot_general(ssz, u8s, (((1,), (0,)), ((), ())),
                          precision=lax.Precision.HIGHEST,
                          preferred_element_type=jnp.float32)

    bstart = lax.broadcasted_iota(jnp.int32, (1, NB), 1).astype(jnp.float32) * BLK
    be = jnp.full((1, NB), float(E - 1), jnp.float32)
    for e in range(E):
        off_e = off[0:1, e:e + 1]
        end_e = off_e + ssz[0:1, e:e + 1]
        be = jnp.where((bstart >= off_e) & (bstart < end_e), float(e), be)
    total = off[0:1, E - 1:E] + ssz[0:1, E - 1:E]
    bv = (bstart < total)
    be_ref[...] = be.astype(jnp.int32)
    bv_ref[...] = bv.astype(jnp.int32)

    lns = (lax.broadcasted_iota(jnp.int32, (N, N), 1) <
           lax.broadcasted_iota(jnp.int32, (N, N), 0)).astype(jnp.float32)
    ex = lax.dot_general(lns, oh, (((1,), (0,)), ((), ())),
                         preferred_element_type=jnp.float32)
    pos0 = jnp.sum(oh1 * (off + ex), axis=1, keepdims=True)
    pos1 = jnp.sum(oh2 * (off + ex + oh1), axis=1, keepdims=True)

    pos_ref[...] = jnp.concatenate([pos0, pos1], axis=1).astype(jnp.int32)
    w_ref[...] = jnp.concatenate([v1, v2], axis=1) * ROUTE_SCALE


def _routing(x, gate_w):
    return pl.pallas_call(
        _routing_body,
        out_shape=[
            jax.ShapeDtypeStruct((N, TOPK), jnp.int32),
            jax.ShapeDtypeStruct((N, TOPK), jnp.float32),
            jax.ShapeDtypeStruct((1, NB), jnp.int32),
            jax.ShapeDtypeStruct((1, NB), jnp.int32),
        ],
    )(x, gate_w)


TCH = 16
NCH = CTOK // TCH


def _sc_dispatch(x, p0, p1, w0, w1):
    mesh = plsc.VectorSubcoreMesh(core_axis_name="c", subcore_axis_name="s",
                                  num_cores=NC)

    @functools.partial(
        pl.kernel, mesh=mesh,
        out_type=[
            jax.ShapeDtypeStruct((P, D), jnp.float32),
            jax.ShapeDtypeStruct((P,), jnp.float32),
        ],
        scratch_types=[
            pltpu.VMEM((CTOK, D), jnp.float32),
            pltpu.VMEM((NCH, TCH), jnp.int32),
            pltpu.VMEM((NCH, TCH), jnp.int32),
            pltpu.VMEM((NCH, TCH), jnp.float32),
            pltpu.VMEM((NCH, TCH), jnp.float32),
            pltpu.SemaphoreType.DMA,
            pltpu.SemaphoreType.DMA,
            pltpu.SemaphoreType.DMA,
            pltpu.SemaphoreType.DMA,
            pltpu.SemaphoreType.DMA,
        ],
    )
    def k(x_hbm, p0_hbm, p1_hbm, w0_hbm, w1_hbm, xs_hbm, wp_hbm,
          rows_v, i0_v, i1_v, w0_v, w1_v, sem, *sem_x):
        wid = lax.axis_index("s") * NC + lax.axis_index("c")
        xh = [pltpu.async_copy(
            x_hbm.at[pl.ds(wid * CTOK + c * TCH, TCH)],
            rows_v.at[pl.ds(c * TCH, TCH)], sem_x[c]) for c in range(NCH)]
        pltpu.sync_copy(p0_hbm.at[wid], i0_v)
        pltpu.sync_copy(p1_hbm.at[wid], i1_v)
        pltpu.sync_copy(w0_hbm.at[wid], w0_v)
        pltpu.sync_copy(w1_hbm.at[wid], w1_v)
        handles = []
        for c in range(NCH):
            xh[c].wait()
            src = rows_v.at[pl.ds(c * TCH, TCH)]
            handles.append(pltpu.async_copy(src, xs_hbm.at[i0_v.at[c]], sem))
            handles.append(pltpu.async_copy(src, xs_hbm.at[i1_v.at[c]], sem))
            handles.append(pltpu.async_copy(w0_v.at[c], wp_hbm.at[i0_v.at[c]],
                                            sem))
            handles.append(pltpu.async_copy(w1_v.at[c], wp_hbm.at[i1_v.at[c]],
                                            sem))
        for h in handles:
            h.wait()

    return k(x, p0, p1, w0, w1)


def _gemm_body(be_ref, bv_ref, xs_ref, wp_ref, w1_ref, b1_ref, w3_ref,
               b3_ref, w2_ref, b2_ref, eo_ref):
    b = pl.program_id(0)

    @pl.when(bv_ref[b] == 1)
    def _():
        xs = xs_ref[...]
        h1 = lax.dot_general(xs, w1_ref[0], (((1,), (1,)), ((), ())),
                             preferred_element_type=jnp.float32) + b1_ref[0]
        h3 = lax.dot_general(xs, w3_ref[0], (((1,), (1,)), ((), ())),
                             preferred_element_type=jnp.float32) + b3_ref[0]
        h = jnp.where(h1 >= 0, h1, 0.01 * h1) * h3
        eo = lax.dot_general(h, w2_ref[0], (((1,), (1,)), ((), ())),
                             preferred_element_type=jnp.float32) + b2_ref[0]
        eo_ref[...] = eo * wp_ref[...]

    @pl.when(bv_ref[b] == 0)
    def _():
        eo_ref[...] = jnp.zeros((BLK, D), jnp.float32)


def _grouped_gemm(xs, wp, w1, b1, w3, b3, w2, b2, be, bv):
    grid_spec = pltpu.PrefetchScalarGridSpec(
        num_scalar_prefetch=2,
        grid=(NB,),
        in_specs=[
            pl.BlockSpec((BLK, D), lambda b, be, bv: (b, 0)),
            pl.BlockSpec((BLK, 1), lambda b, be, bv: (b, 0)),
            pl.BlockSpec((1, I_, D), lambda b, be, bv: (be[b], 0, 0)),
            pl.BlockSpec((1, 1, I_), lambda b, be, bv: (be[b], 0, 0)),
            pl.BlockSpec((1, I_, D), lambda b, be, bv: (be[b], 0, 0)),
            pl.BlockSpec((1, 1, I_), lambda b, be, bv: (be[b], 0, 0)),
            pl.BlockSpec((1, D, I_), lambda b, be, bv: (be[b], 0, 0)),
            pl.BlockSpec((1, 1, D), lambda b, be, bv: (be[b], 0, 0)),
        ],
        out_specs=pl.BlockSpec((BLK, D), lambda b, be, bv: (b, 0)),
    )
    return pl.pallas_call(
        _gemm_body,
        grid_spec=grid_spec,
        out_shape=jax.ShapeDtypeStruct((P, D), jnp.float32),
    )(be, bv, xs, wp, w1, b1.reshape(E, 1, I_), w3, b3.reshape(E, 1, I_),
      w2, b2.reshape(E, 1, D))


def _sc_combine(eo, pos_flat):
    mesh = plsc.VectorSubcoreMesh(core_axis_name="c", subcore_axis_name="s",
                                  num_cores=NC)

    @functools.partial(
        pl.kernel, mesh=mesh,
        out_type=jax.ShapeDtypeStruct((N, D), jnp.float32),
        scratch_types=[
            pltpu.VMEM((TOPK * CTOK,), jnp.int32),
            pltpu.VMEM((TOPK * CCH, D), jnp.float32),
            pltpu.VMEM((TOPK * CCH, D), jnp.float32),
            pltpu.VMEM((CCH, D), jnp.float32),
            pltpu.VMEM((CCH, D), jnp.float32),
            pltpu.SemaphoreType.DMA,
            pltpu.SemaphoreType.DMA,
            pltpu.SemaphoreType.DMA,
            pltpu.SemaphoreType.DMA,
        ],
    )
    def k(eo_hbm, pos_hbm, out_hbm, idx_v, rows_a, rows_b, y_a, y_b,
          sem_a, sem_b, sem_ya, sem_yb):
        wid = lax.axis_index("s") * NC + lax.axis_index("c")
        base = wid * TOPK * CTOK
        nch = CTOK // CCH
        pltpu.sync_copy(pos_hbm.at[pl.ds(base, TOPK * CTOK)], idx_v)
        bufs = [rows_a, rows_b]
        sems = [sem_a, sem_b]
        ybufs = [y_a, y_b]
        ysems = [sem_ya, sem_yb]
        handles = [None] * nch
        yhandles = [None] * nch
        handles[0] = pltpu.async_copy(
            eo_hbm.at[idx_v.at[pl.ds(0, TOPK * CCH)]], bufs[0], sems[0])
        for c in range(nch):
            handles[c].wait()
            if c + 1 < nch:
                handles[c + 1] = pltpu.async_copy(
                    eo_hbm.at[idx_v.at[pl.ds((c + 1) * TOPK * CCH,
                                             TOPK * CCH)]],
                    bufs[(c + 1) % 2], sems[(c + 1) % 2])
            rows_v = bufs[c % 2]
            y_v = ybufs[c % 2]
            if c >= 2:
                yhandles[c - 2].wait()

            def pair_add(t, carry):
                def lane_add(j, carry2):
                    y_v[t, pl.ds(j * 16, 16)] = (
                        rows_v[2 * t, pl.ds(j * 16, 16)] +
                        rows_v[2 * t + 1, pl.ds(j * 16, 16)])
                    return carry2
                return lax.fori_loop(0, D // 16, lane_add, carry)
            lax.fori_loop(0, CCH, pair_add, 0)
            yhandles[c] = pltpu.async_copy(
                y_v, out_hbm.at[pl.ds(wid * CTOK + c * CCH, CCH)],
                ysems[c % 2])
        for c in range(max(0, nch - 2), nch):
            yhandles[c].wait()

    return k(eo, pos_flat)


def _shared_body(x_ref, sw1_ref, sb1_ref, sw3_ref, sb3_ref, sw2_ref, sb2_ref,
                 z_ref):
    x = x_ref[...]
    s1 = lax.dot_general(x, sw1_ref[...], (((1,), (1,)), ((), ())),
                         preferred_element_type=jnp.float32) + sb1_ref[...]
    s3 = lax.dot_general(x, sw3_ref[...], (((1,), (1,)), ((), ())),
                         preferred_element_type=jnp.float32) + sb3_ref[...]
    h = jnp.where(s1 >= 0, s1, 0.01 * s1) * s3
    z_ref[...] = lax.dot_general(h, sw2_ref[...], (((1,), (1,)), ((), ())),
                                 preferred_element_type=jnp.float32) + sb2_ref[...]


def _shared_ffn(x, sw1, sb1, sw3, sb3, sw2, sb2):
    return pl.pallas_call(
        _shared_body,
        grid=(N // SBLK,),
        in_specs=[
            pl.BlockSpec((SBLK, D), lambda b: (b, 0)),
            pl.BlockSpec((SI, D), lambda b: (0, 0)),
            pl.BlockSpec((1, SI), lambda b: (0, 0)),
            pl.BlockSpec((SI, D), lambda b: (0, 0)),
            pl.BlockSpec((1, SI), lambda b: (0, 0)),
            pl.BlockSpec((D, SI), lambda b: (0, 0)),
            pl.BlockSpec((1, D), lambda b: (0, 0)),
        ],
        out_specs=pl.BlockSpec((SBLK, D), lambda b: (b, 0)),
        out_shape=jax.ShapeDtypeStruct((N, D), jnp.float32),
    )(x, sw1, sb1.reshape(1, SI), sw3, sb3.reshape(1, SI), sw2,
      sb2.reshape(1, D))


def _final_body(y_ref, z_ref, ow_ref, ob_ref, out_ref):
    yz = y_ref[...] + z_ref[...]
    out_ref[...] = lax.dot_general(yz, ow_ref[...], (((1,), (1,)), ((), ())),
                                   preferred_element_type=jnp.float32) + ob_ref[...]


def _final(y, z, out_w, out_b):
    return pl.pallas_call(
        _final_body,
        grid=(N // FBLK,),
        in_specs=[
            pl.BlockSpec((FBLK, D), lambda b: (b, 0)),
            pl.BlockSpec((FBLK, D), lambda b: (b, 0)),
            pl.BlockSpec((OUT, D), lambda b: (0, 0)),
            pl.BlockSpec((1, OUT), lambda b: (0, 0)),
        ],
        out_specs=pl.BlockSpec((FBLK, OUT), lambda b: (b, 0)),
        out_shape=jax.ShapeDtypeStruct((N, OUT), jnp.float32),
    )(y, z, out_w, out_b.reshape(1, OUT))


@jax.jit
def _moe(x, gate_w, w1, b1, w2, b2, w3, b3, sw1, sb1, sw2, sb2, sw3, sb3,
         out_w, out_b):
    pos, wts, be, bv = _routing(x, gate_w)
    xs, wp = _sc_dispatch(
        x,
        pos[:, 0].reshape(NW, NCH, TCH), pos[:, 1].reshape(NW, NCH, TCH),
        wts[:, 0].reshape(NW, NCH, TCH), wts[:, 1].reshape(NW, NCH, TCH))
    z = _shared_ffn(x, sw1, sb1, sw3, sb3, sw2, sb2)
    eo = _grouped_gemm(xs, wp.reshape(P, 1), w1, b1, w3, b3, w2, b2,
                       be.reshape(NB), bv.reshape(NB))
    y = _sc_combine(eo, pos.reshape(TOPK * N))
    return _final(y, z, out_w, out_b)


def kernel(x, task_id, gate_w, W1, B1, W2, B2, W3, B3, sw1, sb1, sw2, sb2,
           sw3, sb3, out_w, out_b):
    xf = x.reshape(N, D)
    return _moe(xf, gate_w, W1, B1, W2, B2, W3, B3, sw1, sb1, sw2, sb2,
                sw3, sb3, out_w, out_b)


# speedup vs baseline: 2.2713x; 1.0517x over previous
import functools

import jax
import jax.numpy as jnp
from jax import lax
from jax.experimental import pallas as pl
from jax.experimental.pallas import tpu as pltpu
from jax.experimental.pallas import tpu_sc as plsc

E = 8
TOPK = 2
N = 2048
D = 1024
I_ = 1024
SI = 1024
OUT = 1024
ROUTE_SCALE = 1.0

BLK = 512
NB = (TOPK * N + E * (BLK - 1) + BLK - 1) // BLK
P = NB * BLK
GCH = 512
SBLK = 1024
FBLK = 512

NC = 2
NS = 16
NW = NC * NS
GROWS = P // NW
GCHUNK = 32
CTOK = N // NW
CCH = 16


def _routing_body(x_ref, gw_ref, pos_ref, w_ref, be_ref, bv_ref):
    x = x_ref[...]
    gw = gw_ref[...]
    s = lax.dot_general(x, gw, (((1,), (1,)), ((), ())),
                        preferred_element_type=jnp.float32)
    m = jnp.max(s, axis=1, keepdims=True)
    p = jnp.exp(s - m)
    p = p / jnp.sum(p, axis=1, keepdims=True)

    iota_e = lax.broadcasted_iota(jnp.int32, (N, E), 1)
    v1 = jnp.max(p, axis=1, keepdims=True)
    i1 = jnp.min(jnp.where(p == v1, iota_e, E), axis=1, keepdims=True)
    pm = jnp.where(iota_e == i1, -1.0, p)
    v2 = jnp.max(pm, axis=1, keepdims=True)
    i2 = jnp.min(jnp.where(pm == v2, iota_e, E), axis=1, keepdims=True)

    oh1 = (iota_e == i1).astype(jnp.float32)
    oh2 = (iota_e == i2).astype(jnp.float32)
    oh = oh1 + oh2
    c = jnp.sum(oh, axis=0, keepdims=True)
    ssz = jnp.floor((c + (BLK - 1)) * (1.0 / BLK)) * BLK
    u8s = (lax.broadcasted_iota(jnp.int32, (E, E), 0) <
           lax.broadcasted_iota(jnp.int32, (E, E), 1)).astype(jnp.float32)
    off = lax.dot_general(ssz, u8s, (((1,), (0,)), ((), ())),
                          precision=lax.Precision.HIGHEST,
                          preferred_element_type=jnp.float32)

    bstart = lax.broadcasted_iota(jnp.int32, (1, NB), 1).astype(jnp.float32) * BLK
    be = jnp.full((1, NB), float(E - 1), jnp.float32)
    for e in range(E):
        off_e = off[0:1, e:e + 1]
        end_e = off_e + ssz[0:1, e:e + 1]
        be = jnp.where((bstart >= off_e) & (bstart < end_e), float(e), be)
    total = off[0:1, E - 1:E] + ssz[0:1, E - 1:E]
    bv = (bstart < total)
    be_ref[...] = be.astype(jnp.int32)
    bv_ref[...] = bv.astype(jnp.int32)

    lns = (lax.broadcasted_iota(jnp.int32, (N, N), 1) <
           lax.broadcasted_iota(jnp.int32, (N, N), 0)).astype(jnp.float32)
    ex = lax.dot_general(lns, oh, (((1,), (0,)), ((), ())),
                         preferred_element_type=jnp.float32)
    pos0 = jnp.sum(oh1 * (off + ex), axis=1, keepdims=True)
    pos1 = jnp.sum(oh2 * (off + ex + oh1), axis=1, keepdims=True)

    pos_ref[...] = jnp.concatenate([pos0, pos1], axis=1).astype(jnp.int32)
    w_ref[...] = jnp.concatenate([v1, v2], axis=1) * ROUTE_SCALE


def _routing(x, gate_w):
    return pl.pallas_call(
        _routing_body,
        out_shape=[
            jax.ShapeDtypeStruct((N, TOPK), jnp.int32),
            jax.ShapeDtypeStruct((N, TOPK), jnp.float32),
            jax.ShapeDtypeStruct((1, NB), jnp.int32),
            jax.ShapeDtypeStruct((1, NB), jnp.int32),
        ],
    )(x, gate_w)


TCH = 32
NCH = CTOK // TCH


def _sc_dispatch(x, p0, p1, w0, w1):
    mesh = plsc.VectorSubcoreMesh(core_axis_name="c", subcore_axis_name="s",
                                  num_cores=NC)

    @functools.partial(
        pl.kernel, mesh=mesh,
        out_type=[
            jax.ShapeDtypeStruct((P, D), jnp.float32),
            jax.ShapeDtypeStruct((P,), jnp.float32),
        ],
        scratch_types=[
            pltpu.VMEM((CTOK, D), jnp.float32),
            pltpu.VMEM((NCH, TCH), jnp.int32),
            pltpu.VMEM((NCH, TCH), jnp.int32),
            pltpu.VMEM((NCH, TCH), jnp.float32),
            pltpu.VMEM((NCH, TCH), jnp.float32),
            pltpu.SemaphoreType.DMA,
            pltpu.SemaphoreType.DMA,
            pltpu.SemaphoreType.DMA,
            pltpu.SemaphoreType.DMA,
            pltpu.SemaphoreType.DMA,
        ],
    )
    def k(x_hbm, p0_hbm, p1_hbm, w0_hbm, w1_hbm, xs_hbm, wp_hbm,
          rows_v, i0_v, i1_v, w0_v, w1_v, sem, *sem_x):
        wid = lax.axis_index("s") * NC + lax.axis_index("c")
        xh = [pltpu.async_copy(
            x_hbm.at[pl.ds(wid * CTOK + c * TCH, TCH)],
            rows_v.at[pl.ds(c * TCH, TCH)], sem_x[c]) for c in range(NCH)]
        pltpu.sync_copy(p0_hbm.at[wid], i0_v)
        pltpu.sync_copy(p1_hbm.at[wid], i1_v)
        pltpu.sync_copy(w0_hbm.at[wid], w0_v)
        pltpu.sync_copy(w1_hbm.at[wid], w1_v)
        handles = []
        for c in range(NCH):
            xh[c].wait()
            src = rows_v.at[pl.ds(c * TCH, TCH)]
            handles.append(pltpu.async_copy(src, xs_hbm.at[i0_v.at[c]], sem))
            handles.append(pltpu.async_copy(src, xs_hbm.at[i1_v.at[c]], sem))
            handles.append(pltpu.async_copy(w0_v.at[c], wp_hbm.at[i0_v.at[c]],
                                            sem))
            handles.append(pltpu.async_copy(w1_v.at[c], wp_hbm.at[i1_v.at[c]],
                                            sem))
        for h in handles:
            h.wait()

    return k(x, p0, p1, w0, w1)


def _gemm_body(be_ref, bv_ref, xs_ref, wp_ref, w1_ref, b1_ref, w3_ref,
               b3_ref, w2_ref, b2_ref, eo_ref):
    b = pl.program_id(0)

    @pl.when(bv_ref[b] == 1)
    def _():
        xs = xs_ref[...]
        h1 = lax.dot_general(xs, w1_ref[0], (((1,), (1,)), ((), ())),
                             preferred_element_type=jnp.float32) + b1_ref[0]
        h3 = lax.dot_general(xs, w3_ref[0], (((1,), (1,)), ((), ())),
                             preferred_element_type=jnp.float32) + b3_ref[0]
        h = jnp.where(h1 >= 0, h1, 0.01 * h1) * h3
        eo = lax.dot_general(h, w2_ref[0], (((1,), (1,)), ((), ())),
                             preferred_element_type=jnp.float32) + b2_ref[0]
        eo_ref[...] = eo * wp_ref[...]

    @pl.when(bv_ref[b] == 0)
    def _():
        eo_ref[...] = jnp.zeros((BLK, D), jnp.float32)


def _grouped_gemm(xs, wp, w1, b1, w3, b3, w2, b2, be, bv):
    grid_spec = pltpu.PrefetchScalarGridSpec(
        num_scalar_prefetch=2,
        grid=(NB,),
        in_specs=[
            pl.BlockSpec((BLK, D), lambda b, be, bv: (b, 0)),
            pl.BlockSpec((BLK, 1), lambda b, be, bv: (b, 0)),
            pl.BlockSpec((1, I_, D), lambda b, be, bv: (be[b], 0, 0)),
            pl.BlockSpec((1, 1, I_), lambda b, be, bv: (be[b], 0, 0)),
            pl.BlockSpec((1, I_, D), lambda b, be, bv: (be[b], 0, 0)),
            pl.BlockSpec((1, 1, I_), lambda b, be, bv: (be[b], 0, 0)),
            pl.BlockSpec((1, D, I_), lambda b, be, bv: (be[b], 0, 0)),
            pl.BlockSpec((1, 1, D), lambda b, be, bv: (be[b], 0, 0)),
        ],
        out_specs=pl.BlockSpec((BLK, D), lambda b, be, bv: (b, 0)),
    )
    return pl.pallas_call(
        _gemm_body,
        grid_spec=grid_spec,
        out_shape=jax.ShapeDtypeStruct((P, D), jnp.float32),
    )(be, bv, xs, wp, w1, b1.reshape(E, 1, I_), w3, b3.reshape(E, 1, I_),
      w2, b2.reshape(E, 1, D))


def _sc_combine(eo, pos_flat):
    mesh = plsc.VectorSubcoreMesh(core_axis_name="c", subcore_axis_name="s",
                                  num_cores=NC)

    @functools.partial(
        pl.kernel, mesh=mesh,
        out_type=jax.ShapeDtypeStruct((N, D), jnp.float32),
        scratch_types=[
            pltpu.VMEM((TOPK * CTOK,), jnp.int32),
            pltpu.VMEM((TOPK * CCH, D), jnp.float32),
            pltpu.VMEM((TOPK * CCH, D), jnp.float32),
            pltpu.VMEM((CCH, D), jnp.float32),
            pltpu.VMEM((CCH, D), jnp.float32),
            pltpu.SemaphoreType.DMA,
            pltpu.SemaphoreType.DMA,
            pltpu.SemaphoreType.DMA,
            pltpu.SemaphoreType.DMA,
        ],
    )
    def k(eo_hbm, pos_hbm, out_hbm, idx_v, rows_a, rows_b, y_a, y_b,
          sem_a, sem_b, sem_ya, sem_yb):
        wid = lax.axis_index("s") * NC + lax.axis_index("c")
        base = wid * TOPK * CTOK
        nch = CTOK // CCH
        pltpu.sync_copy(pos_hbm.at[pl.ds(base, TOPK * CTOK)], idx_v)
        bufs = [rows_a, rows_b]
        sems = [sem_a, sem_b]
        ybufs = [y_a, y_b]
        ysems = [sem_ya, sem_yb]
        handles = [None] * nch
        yhandles = [None] * nch
        handles[0] = pltpu.async_copy(
            eo_hbm.at[idx_v.at[pl.ds(0, TOPK * CCH)]], bufs[0], sems[0])
        for c in range(nch):
            handles[c].wait()
            if c + 1 < nch:
                handles[c + 1] = pltpu.async_copy(
                    eo_hbm.at[idx_v.at[pl.ds((c + 1) * TOPK * CCH,
                                             TOPK * CCH)]],
                    bufs[(c + 1) % 2], sems[(c + 1) % 2])
            rows_v = bufs[c % 2]
            y_v = ybufs[c % 2]
            if c >= 2:
                yhandles[c - 2].wait()

            def pair_add(t, carry):
                def lane_add(j, carry2):
                    y_v[t, pl.ds(j * 16, 16)] = (
                        rows_v[2 * t, pl.ds(j * 16, 16)] +
                        rows_v[2 * t + 1, pl.ds(j * 16, 16)])
                    return carry2
                return lax.fori_loop(0, D // 16, lane_add, carry, unroll=8)
            lax.fori_loop(0, CCH, pair_add, 0)
            yhandles[c] = pltpu.async_copy(
                y_v, out_hbm.at[pl.ds(wid * CTOK + c * CCH, CCH)],
                ysems[c % 2])
        for c in range(max(0, nch - 2), nch):
            yhandles[c].wait()

    return k(eo, pos_flat)


def _shared_body(x_ref, sw1_ref, sb1_ref, sw3_ref, sb3_ref, sw2_ref, sb2_ref,
                 z_ref):
    x = x_ref[...]
    s1 = lax.dot_general(x, sw1_ref[...], (((1,), (1,)), ((), ())),
                         preferred_element_type=jnp.float32) + sb1_ref[...]
    s3 = lax.dot_general(x, sw3_ref[...], (((1,), (1,)), ((), ())),
                         preferred_element_type=jnp.float32) + sb3_ref[...]
    h = jnp.where(s1 >= 0, s1, 0.01 * s1) * s3
    z_ref[...] = lax.dot_general(h, sw2_ref[...], (((1,), (1,)), ((), ())),
                                 preferred_element_type=jnp.float32) + sb2_ref[...]


def _shared_ffn(x, sw1, sb1, sw3, sb3, sw2, sb2):
    return pl.pallas_call(
        _shared_body,
        grid=(N // SBLK,),
        in_specs=[
            pl.BlockSpec((SBLK, D), lambda b: (b, 0)),
            pl.BlockSpec((SI, D), lambda b: (0, 0)),
            pl.BlockSpec((1, SI), lambda b: (0, 0)),
            pl.BlockSpec((SI, D), lambda b: (0, 0)),
            pl.BlockSpec((1, SI), lambda b: (0, 0)),
            pl.BlockSpec((D, SI), lambda b: (0, 0)),
            pl.BlockSpec((1, D), lambda b: (0, 0)),
        ],
        out_specs=pl.BlockSpec((SBLK, D), lambda b: (b, 0)),
        out_shape=jax.ShapeDtypeStruct((N, D), jnp.float32),
    )(x, sw1, sb1.reshape(1, SI), sw3, sb3.reshape(1, SI), sw2,
      sb2.reshape(1, D))


def _final_body(y_ref, z_ref, ow_ref, ob_ref, out_ref):
    yz = y_ref[...] + z_ref[...]
    out_ref[...] = lax.dot_general(yz, ow_ref[...], (((1,), (1,)), ((), ())),
                                   preferred_element_type=jnp.float32) + ob_ref[...]


def _final(y, z, out_w, out_b):
    return pl.pallas_call(
        _final_body,
        grid=(N // FBLK,),
        in_specs=[
            pl.BlockSpec((FBLK, D), lambda b: (b, 0)),
            pl.BlockSpec((FBLK, D), lambda b: (b, 0)),
            pl.BlockSpec((OUT, D), lambda b: (0, 0)),
            pl.BlockSpec((1, OUT), lambda b: (0, 0)),
        ],
        out_specs=pl.BlockSpec((FBLK, OUT), lambda b: (b, 0)),
        out_shape=jax.ShapeDtypeStruct((N, OUT), jnp.float32),
    )(y, z, out_w, out_b.reshape(1, OUT))


@jax.jit
def _moe(x, gate_w, w1, b1, w2, b2, w3, b3, sw1, sb1, sw2, sb2, sw3, sb3,
         out_w, out_b):
    pos, wts, be, bv = _routing(x, gate_w)
    xs, wp = _sc_dispatch(
        x,
        pos[:, 0].reshape(NW, NCH, TCH), pos[:, 1].reshape(NW, NCH, TCH),
        wts[:, 0].reshape(NW, NCH, TCH), wts[:, 1].reshape(NW, NCH, TCH))
    z = _shared_ffn(x, sw1, sb1, sw3, sb3, sw2, sb2)
    eo = _grouped_gemm(xs, wp.reshape(P, 1), w1, b1, w3, b3, w2, b2,
                       be.reshape(NB), bv.reshape(NB))
    y = _sc_combine(eo, pos.reshape(TOPK * N))
    return _final(y, z, out_w, out_b)


def kernel(x, task_id, gate_w, W1, B1, W2, B2, W3, B3, sw1, sb1, sw2, sb2,
           sw3, sb3, out_w, out_b):
    xf = x.reshape(N, D)
    return _moe(xf, gate_w, W1, B1, W2, B2, W3, B3, sw1, sb1, sw2, sb2,
                sw3, sb3, out_w, out_b)


# speedup vs baseline: 2.2731x; 1.0008x over previous
import functools

import jax
import jax.numpy as jnp
from jax import lax
from jax.experimental import pallas as pl
from jax.experimental.pallas import tpu as pltpu
from jax.experimental.pallas import tpu_sc as plsc

E = 8
TOPK = 2
N = 2048
D = 1024
I_ = 1024
SI = 1024
OUT = 1024
ROUTE_SCALE = 1.0

BLK = 512
NB = (TOPK * N + E * (BLK - 1) + BLK - 1) // BLK
P = NB * BLK
GCH = 512
SBLK = 1024
FBLK = 512

NC = 2
NS = 16
NW = NC * NS
GROWS = P // NW
GCHUNK = 32
CTOK = N // NW
CCH = 16


def _routing_body(x_ref, gw_ref, pos_ref, w_ref, be_ref, bv_ref):
    x = x_ref[...]
    gw = gw_ref[...]
    s = lax.dot_general(x, gw, (((1,), (1,)), ((), ())),
                        preferred_element_type=jnp.float32)
    m = jnp.max(s, axis=1, keepdims=True)
    p = jnp.exp(s - m)
    p = p / jnp.sum(p, axis=1, keepdims=True)

    iota_e = lax.broadcasted_iota(jnp.int32, (N, E), 1)
    v1 = jnp.max(p, axis=1, keepdims=True)
    i1 = jnp.min(jnp.where(p == v1, iota_e, E), axis=1, keepdims=True)
    pm = jnp.where(iota_e == i1, -1.0, p)
    v2 = jnp.max(pm, axis=1, keepdims=True)
    i2 = jnp.min(jnp.where(pm == v2, iota_e, E), axis=1, keepdims=True)

    oh1 = (iota_e == i1).astype(jnp.float32)
    oh2 = (iota_e == i2).astype(jnp.float32)
    oh = oh1 + oh2
    c = jnp.sum(oh, axis=0, keepdims=True)
    ssz = jnp.floor((c + (BLK - 1)) * (1.0 / BLK)) * BLK
    u8s = (lax.broadcasted_iota(jnp.int32, (E, E), 0) <
           lax.broadcasted_iota(jnp.int32, (E, E), 1)).astype(jnp.float32)
    off = lax.dot_general(ssz, u8s, (((1,), (0,)), ((), ())),
                          precision=lax.Precision.HIGHEST,
                          preferred_element_type=jnp.float32)

    bstart = lax.broadcasted_iota(jnp.int32, (1, NB), 1).astype(jnp.float32) * BLK
    be = jnp.full((1, NB), float(E - 1), jnp.float32)
    for e in range(E):
        off_e = off[0:1, e:e + 1]
        end_e = off_e + ssz[0:1, e:e + 1]
        be = jnp.where((bstart >= off_e) & (bstart < end_e), float(e), be)
    total = off[0:1, E - 1:E] + ssz[0:1, E - 1:E]
    bv = (bstart < total)
    be_ref[...] = be.astype(jnp.int32)
    bv_ref[...] = bv.astype(jnp.int32)

    lns = (lax.broadcasted_iota(jnp.int32, (N, N), 1) <
           lax.broadcasted_iota(jnp.int32, (N, N), 0)).astype(jnp.float32)
    ex = lax.dot_general(lns, oh, (((1,), (0,)), ((), ())),
                         preferred_element_type=jnp.float32)
    pos0 = jnp.sum(oh1 * (off + ex), axis=1, keepdims=True)
    pos1 = jnp.sum(oh2 * (off + ex + oh1), axis=1, keepdims=True)

    pos_ref[...] = jnp.concatenate([pos0, pos1], axis=1).astype(jnp.int32)
    w_ref[...] = jnp.concatenate([v1, v2], axis=1) * ROUTE_SCALE


def _routing(x, gate_w):
    return pl.pallas_call(
        _routing_body,
        out_shape=[
            jax.ShapeDtypeStruct((N, TOPK), jnp.int32),
            jax.ShapeDtypeStruct((N, TOPK), jnp.float32),
            jax.ShapeDtypeStruct((1, NB), jnp.int32),
            jax.ShapeDtypeStruct((1, NB), jnp.int32),
        ],
    )(x, gate_w)


TCH = 64
NCH = CTOK // TCH


def _sc_dispatch(x, p0, p1, w0, w1):
    mesh = plsc.VectorSubcoreMesh(core_axis_name="c", subcore_axis_name="s",
                                  num_cores=NC)

    @functools.partial(
        pl.kernel, mesh=mesh,
        out_type=[
            jax.ShapeDtypeStruct((P, D), jnp.float32),
            jax.ShapeDtypeStruct((P,), jnp.float32),
        ],
        scratch_types=[
            pltpu.VMEM((CTOK, D), jnp.float32),
            pltpu.VMEM((NCH, TCH), jnp.int32),
            pltpu.VMEM((NCH, TCH), jnp.int32),
            pltpu.VMEM((NCH, TCH), jnp.float32),
            pltpu.VMEM((NCH, TCH), jnp.float32),
            pltpu.SemaphoreType.DMA,
            pltpu.SemaphoreType.DMA,
            pltpu.SemaphoreType.DMA,
            pltpu.SemaphoreType.DMA,
            pltpu.SemaphoreType.DMA,
        ],
    )
    def k(x_hbm, p0_hbm, p1_hbm, w0_hbm, w1_hbm, xs_hbm, wp_hbm,
          rows_v, i0_v, i1_v, w0_v, w1_v, sem, *sem_x):
        wid = lax.axis_index("s") * NC + lax.axis_index("c")
        xh = [pltpu.async_copy(
            x_hbm.at[pl.ds(wid * CTOK + c * TCH, TCH)],
            rows_v.at[pl.ds(c * TCH, TCH)], sem_x[c]) for c in range(NCH)]
        pltpu.sync_copy(p0_hbm.at[wid], i0_v)
        pltpu.sync_copy(p1_hbm.at[wid], i1_v)
        pltpu.sync_copy(w0_hbm.at[wid], w0_v)
        pltpu.sync_copy(w1_hbm.at[wid], w1_v)
        handles = []
        for c in range(NCH):
            xh[c].wait()
            src = rows_v.at[pl.ds(c * TCH, TCH)]
            handles.append(pltpu.async_copy(src, xs_hbm.at[i0_v.at[c]], sem))
            handles.append(pltpu.async_copy(src, xs_hbm.at[i1_v.at[c]], sem))
            handles.append(pltpu.async_copy(w0_v.at[c], wp_hbm.at[i0_v.at[c]],
                                            sem))
            handles.append(pltpu.async_copy(w1_v.at[c], wp_hbm.at[i1_v.at[c]],
                                            sem))
        for h in handles:
            h.wait()

    return k(x, p0, p1, w0, w1)


def _gemm_body(be_ref, bv_ref, xs_ref, wp_ref, w1_ref, b1_ref, w3_ref,
               b3_ref, w2_ref, b2_ref, eo_ref):
    b = pl.program_id(0)

    @pl.when(bv_ref[b] == 1)
    def _():
        xs = xs_ref[...]
        h1 = lax.dot_general(xs, w1_ref[0], (((1,), (1,)), ((), ())),
                             preferred_element_type=jnp.float32) + b1_ref[0]
        h3 = lax.dot_general(xs, w3_ref[0], (((1,), (1,)), ((), ())),
                             preferred_element_type=jnp.float32) + b3_ref[0]
        h = jnp.where(h1 >= 0, h1, 0.01 * h1) * h3
        eo = lax.dot_general(h, w2_ref[0], (((1,), (1,)), ((), ())),
                             preferred_element_type=jnp.float32) + b2_ref[0]
        eo_ref[...] = eo * wp_ref[...]

    @pl.when(bv_ref[b] == 0)
    def _():
        eo_ref[...] = jnp.zeros((BLK, D), jnp.float32)


def _grouped_gemm(xs, wp, w1, b1, w3, b3, w2, b2, be, bv):
    grid_spec = pltpu.PrefetchScalarGridSpec(
        num_scalar_prefetch=2,
        grid=(NB,),
        in_specs=[
            pl.BlockSpec((BLK, D), lambda b, be, bv: (b, 0)),
            pl.BlockSpec((BLK, 1), lambda b, be, bv: (b, 0)),
            pl.BlockSpec((1, I_, D), lambda b, be, bv: (be[b], 0, 0)),
            pl.BlockSpec((1, 1, I_), lambda b, be, bv: (be[b], 0, 0)),
            pl.BlockSpec((1, I_, D), lambda b, be, bv: (be[b], 0, 0)),
            pl.BlockSpec((1, 1, I_), lambda b, be, bv: (be[b], 0, 0)),
            pl.BlockSpec((1, D, I_), lambda b, be, bv: (be[b], 0, 0)),
            pl.BlockSpec((1, 1, D), lambda b, be, bv: (be[b], 0, 0)),
        ],
        out_specs=pl.BlockSpec((BLK, D), lambda b, be, bv: (b, 0)),
    )
    return pl.pallas_call(
        _gemm_body,
        grid_spec=grid_spec,
        out_shape=jax.ShapeDtypeStruct((P, D), jnp.float32),
    )(be, bv, xs, wp, w1, b1.reshape(E, 1, I_), w3, b3.reshape(E, 1, I_),
      w2, b2.reshape(E, 1, D))


def _sc_combine(eo, pos_flat):
    mesh = plsc.VectorSubcoreMesh(core_axis_name="c", subcore_axis_name="s",
                                  num_cores=NC)

    @functools.partial(
        pl.kernel, mesh=mesh,
        out_type=jax.ShapeDtypeStruct((N, D), jnp.float32),
        scratch_types=[
            pltpu.VMEM((TOPK * CTOK,), jnp.int32),
            pltpu.VMEM((TOPK * CCH, D), jnp.float32),
            pltpu.VMEM((TOPK * CCH, D), jnp.float32),
            pltpu.VMEM((CCH, D), jnp.float32),
            pltpu.VMEM((CCH, D), jnp.float32),
            pltpu.SemaphoreType.DMA,
            pltpu.SemaphoreType.DMA,
            pltpu.SemaphoreType.DMA,
            pltpu.SemaphoreType.DMA,
        ],
    )
    def k(eo_hbm, pos_hbm, out_hbm, idx_v, rows_a, rows_b, y_a, y_b,
          sem_a, sem_b, sem_ya, sem_yb):
        wid = lax.axis_index("s") * NC + lax.axis_index("c")
        base = wid * TOPK * CTOK
        nch = CTOK // CCH
        pltpu.sync_copy(pos_hbm.at[pl.ds(base, TOPK * CTOK)], idx_v)
        bufs = [rows_a, rows_b]
        sems = [sem_a, sem_b]
        ybufs = [y_a, y_b]
        ysems = [sem_ya, sem_yb]
        handles = [None] * nch
        yhandles = [None] * nch
        handles[0] = pltpu.async_copy(
            eo_hbm.at[idx_v.at[pl.ds(0, TOPK * CCH)]], bufs[0], sems[0])
        for c in range(nch):
            handles[c].wait()
            if c + 1 < nch:
                handles[c + 1] = pltpu.async_copy(
                    eo_hbm.at[idx_v.at[pl.ds((c + 1) * TOPK * CCH,
                                             TOPK * CCH)]],
                    bufs[(c + 1) % 2], sems[(c + 1) % 2])
            rows_v = bufs[c % 2]
            y_v = ybufs[c % 2]
            if c >= 2:
                yhandles[c - 2].wait()

            def pair_add(t, carry):
                def lane_add(j, carry2):
                    y_v[t, pl.ds(j * 16, 16)] = (
                        rows_v[2 * t, pl.ds(j * 16, 16)] +
                        rows_v[2 * t + 1, pl.ds(j * 16, 16)])
                    return carry2
                return lax.fori_loop(0, D // 16, lane_add, carry, unroll=8)
            lax.fori_loop(0, CCH, pair_add, 0)
            yhandles[c] = pltpu.async_copy(
                y_v, out_hbm.at[pl.ds(wid * CTOK + c * CCH, CCH)],
                ysems[c % 2])
        for c in range(max(0, nch - 2), nch):
            yhandles[c].wait()

    return k(eo, pos_flat)


def _shared_body(x_ref, sw1_ref, sb1_ref, sw3_ref, sb3_ref, sw2_ref, sb2_ref,
                 z_ref):
    x = x_ref[...]
    s1 = lax.dot_general(x, sw1_ref[...], (((1,), (1,)), ((), ())),
                         preferred_element_type=jnp.float32) + sb1_ref[...]
    s3 = lax.dot_general(x, sw3_ref[...], (((1,), (1,)), ((), ())),
                         preferred_element_type=jnp.float32) + sb3_ref[...]
    h = jnp.where(s1 >= 0, s1, 0.01 * s1) * s3
    z_ref[...] = lax.dot_general(h, sw2_ref[...], (((1,), (1,)), ((), ())),
                                 preferred_element_type=jnp.float32) + sb2_ref[...]


def _shared_ffn(x, sw1, sb1, sw3, sb3, sw2, sb2):
    return pl.pallas_call(
        _shared_body,
        grid=(N // SBLK,),
        in_specs=[
            pl.BlockSpec((SBLK, D), lambda b: (b, 0)),
            pl.BlockSpec((SI, D), lambda b: (0, 0)),
            pl.BlockSpec((1, SI), lambda b: (0, 0)),
            pl.BlockSpec((SI, D), lambda b: (0, 0)),
            pl.BlockSpec((1, SI), lambda b: (0, 0)),
            pl.BlockSpec((D, SI), lambda b: (0, 0)),
            pl.BlockSpec((1, D), lambda b: (0, 0)),
        ],
        out_specs=pl.BlockSpec((SBLK, D), lambda b: (b, 0)),
        out_shape=jax.ShapeDtypeStruct((N, D), jnp.float32),
    )(x, sw1, sb1.reshape(1, SI), sw3, sb3.reshape(1, SI), sw2,
      sb2.reshape(1, D))


def _final_body(y_ref, z_ref, ow_ref, ob_ref, out_ref):
    yz = y_ref[...] + z_ref[...]
    out_ref[...] = lax.dot_general(yz, ow_ref[...], (((1,), (1,)), ((), ())),
                                   preferred_element_type=jnp.float32) + ob_ref[...]


def _final(y, z, out_w, out_b):
    return pl.pallas_call(
        _final_body,
        grid=(N // FBLK,),
        in_specs=[
            pl.BlockSpec((FBLK, D), lambda b: (b, 0)),
            pl.BlockSpec((FBLK, D), lambda b: (b, 0)),
            pl.BlockSpec((OUT, D), lambda b: (0, 0)),
            pl.BlockSpec((1, OUT), lambda b: (0, 0)),
        ],
        out_specs=pl.BlockSpec((FBLK, OUT), lambda b: (b, 0)),
        out_shape=jax.ShapeDtypeStruct((N, OUT), jnp.float32),
    )(y, z, out_w, out_b.reshape(1, OUT))


@jax.jit
def _moe(x, gate_w, w1, b1, w2, b2, w3, b3, sw1, sb1, sw2, sb2, sw3, sb3,
         out_w, out_b):
    pos, wts, be, bv = _routing(x, gate_w)
    xs, wp = _sc_dispatch(
        x,
        pos[:, 0].reshape(NW, NCH, TCH), pos[:, 1].reshape(NW, NCH, TCH),
        wts[:, 0].reshape(NW, NCH, TCH), wts[:, 1].reshape(NW, NCH, TCH))
    z = _shared_ffn(x, sw1, sb1, sw3, sb3, sw2, sb2)
    eo = _grouped_gemm(xs, wp.reshape(P, 1), w1, b1, w3, b3, w2, b2,
                       be.reshape(NB), bv.reshape(NB))
    y = _sc_combine(eo, pos.reshape(TOPK * N))
    return _final(y, z, out_w, out_b)


def kernel(x, task_id, gate_w, W1, B1, W2, B2, W3, B3, sw1, sb1, sw2, sb2,
           sw3, sb3, out_w, out_b):
    xf = x.reshape(N, D)
    return _moe(xf, gate_w, W1, B1, W2, B2, W3, B3, sw1, sb1, sw2, sb2,
                sw3, sb3, out_w, out_b)


# speedup vs baseline: 2.2743x; 1.0006x over previous
import functools

import jax
import jax.numpy as jnp
from jax import lax
from jax.experimental import pallas as pl
from jax.experimental.pallas import tpu as pltpu
from jax.experimental.pallas import tpu_sc as plsc

E = 8
TOPK = 2
N = 2048
D = 1024
I_ = 1024
SI = 1024
OUT = 1024
ROUTE_SCALE = 1.0

BLK = 512
NB = (TOPK * N + E * (BLK - 1) + BLK - 1) // BLK
P = NB * BLK
SBLK = 1024
FBLK = 512

NC = 2
NS = 16
NW = NC * NS
CTOK = N // NW
CCH = 16


def _routing_body(x_ref, gw_ref, pos_ref, w_ref, be_ref, bv_ref):
    x = x_ref[...]
    gw = gw_ref[...]
    s = lax.dot_general(x, gw, (((1,), (1,)), ((), ())),
                        preferred_element_type=jnp.float32)
    m = jnp.max(s, axis=1, keepdims=True)
    p = jnp.exp(s - m)
    p = p / jnp.sum(p, axis=1, keepdims=True)

    iota_e = lax.broadcasted_iota(jnp.int32, (N, E), 1)
    v1 = jnp.max(p, axis=1, keepdims=True)
    i1 = jnp.min(jnp.where(p == v1, iota_e, E), axis=1, keepdims=True)
    pm = jnp.where(iota_e == i1, -1.0, p)
    v2 = jnp.max(pm, axis=1, keepdims=True)
    i2 = jnp.min(jnp.where(pm == v2, iota_e, E), axis=1, keepdims=True)

    oh1 = (iota_e == i1).astype(jnp.float32)
    oh2 = (iota_e == i2).astype(jnp.float32)
    oh = oh1 + oh2
    c = jnp.sum(oh, axis=0, keepdims=True)
    ssz = jnp.floor((c + (BLK - 1)) * (1.0 / BLK)) * BLK
    u8s = (lax.broadcasted_iota(jnp.int32, (E, E), 0) <
           lax.broadcasted_iota(jnp.int32, (E, E), 1)).astype(jnp.float32)
    off = lax.dot_general(ssz, u8s, (((1,), (0,)), ((), ())),
                          precision=lax.Precision.HIGHEST,
                          preferred_element_type=jnp.float32)

    bstart = lax.broadcasted_iota(jnp.int32, (1, NB), 1).astype(jnp.float32) * BLK
    be = jnp.full((1, NB), float(E - 1), jnp.float32)
    for e in range(E):
        off_e = off[0:1, e:e + 1]
        end_e = off_e + ssz[0:1, e:e + 1]
        be = jnp.where((bstart >= off_e) & (bstart < end_e), float(e), be)
    total = off[0:1, E - 1:E] + ssz[0:1, E - 1:E]
    bv = (bstart < total)
    be_ref[...] = be.astype(jnp.int32)
    bv_ref[...] = bv.astype(jnp.int32)

    lns = (lax.broadcasted_iota(jnp.int32, (N, N), 1) <
           lax.broadcasted_iota(jnp.int32, (N, N), 0)).astype(jnp.float32)
    ex = lax.dot_general(lns, oh, (((1,), (0,)), ((), ())),
                         preferred_element_type=jnp.float32)
    pos0 = jnp.sum(oh1 * (off + ex), axis=1, keepdims=True)
    pos1 = jnp.sum(oh2 * (off + ex + oh1), axis=1, keepdims=True)

    pos_ref[...] = jnp.concatenate([pos0, pos1], axis=1).astype(jnp.int32)
    w_ref[...] = jnp.concatenate([v1, v2], axis=1) * ROUTE_SCALE


def _routing(x, gate_w):
    return pl.pallas_call(
        _routing_body,
        out_shape=[
            jax.ShapeDtypeStruct((N, TOPK), jnp.int32),
            jax.ShapeDtypeStruct((N, TOPK), jnp.float32),
            jax.ShapeDtypeStruct((1, NB), jnp.int32),
            jax.ShapeDtypeStruct((1, NB), jnp.int32),
        ],
    )(x, gate_w)


TCH = 64
NCH = CTOK // TCH


def _sc_dispatch(x, p0, p1, w0, w1):
    mesh = plsc.VectorSubcoreMesh(core_axis_name="c", subcore_axis_name="s",
                                  num_cores=NC)

    @functools.partial(
        pl.kernel, mesh=mesh,
        out_type=[
            jax.ShapeDtypeStruct((P, D), jnp.float32),
            jax.ShapeDtypeStruct((P,), jnp.float32),
        ],
        scratch_types=[
            pltpu.VMEM((CTOK, D), jnp.float32),
            pltpu.VMEM((NCH, TCH), jnp.int32),
            pltpu.VMEM((NCH, TCH), jnp.int32),
            pltpu.VMEM((NCH, TCH), jnp.float32),
            pltpu.VMEM((NCH, TCH), jnp.float32),
            pltpu.SemaphoreType.DMA,
            pltpu.SemaphoreType.DMA,
            pltpu.SemaphoreType.DMA,
            pltpu.SemaphoreType.DMA,
            pltpu.SemaphoreType.DMA,
        ],
    )
    def k(x_hbm, p0_hbm, p1_hbm, w0_hbm, w1_hbm, xs_hbm, wp_hbm,
          rows_v, i0_v, i1_v, w0_v, w1_v, sem, *sem_x):
        wid = lax.axis_index("s") * NC + lax.axis_index("c")
        xh = [pltpu.async_copy(
            x_hbm.at[pl.ds(wid * CTOK + c * TCH, TCH)],
            rows_v.at[pl.ds(c * TCH, TCH)], sem_x[c]) for c in range(NCH)]
        pltpu.sync_copy(p0_hbm.at[wid], i0_v)
        pltpu.sync_copy(p1_hbm.at[wid], i1_v)
        pltpu.sync_copy(w0_hbm.at[wid], w0_v)
        pltpu.sync_copy(w1_hbm.at[wid], w1_v)
        handles = []
        for c in range(NCH):
            xh[c].wait()
            src = rows_v.at[pl.ds(c * TCH, TCH)]
            handles.append(pltpu.async_copy(src, xs_hbm.at[i0_v.at[c]], sem))
            handles.append(pltpu.async_copy(src, xs_hbm.at[i1_v.at[c]], sem))
            handles.append(pltpu.async_copy(w0_v.at[c], wp_hbm.at[i0_v.at[c]],
                                            sem))
            handles.append(pltpu.async_copy(w1_v.at[c], wp_hbm.at[i1_v.at[c]],
                                            sem))
        for h in handles:
            h.wait()

    return k(x, p0, p1, w0, w1)


def _gemm_body(be_ref, bv_ref, xs_ref, wp_ref, w1_ref, b1_ref, w3_ref,
               b3_ref, w2_ref, b2_ref, eo_ref):
    b = pl.program_id(0)

    @pl.when(bv_ref[b] == 1)
    def _():
        xs = xs_ref[...]
        h1 = lax.dot_general(xs, w1_ref[0], (((1,), (1,)), ((), ())),
                             preferred_element_type=jnp.float32) + b1_ref[0]
        h3 = lax.dot_general(xs, w3_ref[0], (((1,), (1,)), ((), ())),
                             preferred_element_type=jnp.float32) + b3_ref[0]
        h = jnp.where(h1 >= 0, h1, 0.01 * h1) * h3
        eo = lax.dot_general(h, w2_ref[0], (((1,), (1,)), ((), ())),
                             preferred_element_type=jnp.float32) + b2_ref[0]
        eo_ref[...] = eo * wp_ref[...]

    @pl.when(bv_ref[b] == 0)
    def _():
        eo_ref[...] = jnp.zeros((BLK, D), jnp.float32)


def _grouped_gemm(xs, wp, w1, b1, w3, b3, w2, b2, be, bv):
    grid_spec = pltpu.PrefetchScalarGridSpec(
        num_scalar_prefetch=2,
        grid=(NB,),
        in_specs=[
            pl.BlockSpec((BLK, D), lambda b, be, bv: (b, 0)),
            pl.BlockSpec((BLK, 1), lambda b, be, bv: (b, 0)),
            pl.BlockSpec((1, I_, D), lambda b, be, bv: (be[b], 0, 0)),
            pl.BlockSpec((1, 1, I_), lambda b, be, bv: (be[b], 0, 0)),
            pl.BlockSpec((1, I_, D), lambda b, be, bv: (be[b], 0, 0)),
            pl.BlockSpec((1, 1, I_), lambda b, be, bv: (be[b], 0, 0)),
            pl.BlockSpec((1, D, I_), lambda b, be, bv: (be[b], 0, 0)),
            pl.BlockSpec((1, 1, D), lambda b, be, bv: (be[b], 0, 0)),
        ],
        out_specs=pl.BlockSpec((BLK, D), lambda b, be, bv: (b, 0)),
    )
    return pl.pallas_call(
        _gemm_body,
        grid_spec=grid_spec,
        out_shape=jax.ShapeDtypeStruct((P, D), jnp.float32),
    )(be, bv, xs, wp, w1, b1.reshape(E, 1, I_), w3, b3.reshape(E, 1, I_),
      w2, b2.reshape(E, 1, D))


def _sc_combine(eo, pos_flat):
    mesh = plsc.VectorSubcoreMesh(core_axis_name="c", subcore_axis_name="s",
                                  num_cores=NC)

    @functools.partial(
        pl.kernel, mesh=mesh,
        out_type=jax.ShapeDtypeStruct((N, D), jnp.float32),
        scratch_types=[
            pltpu.VMEM((TOPK * CTOK,), jnp.int32),
            pltpu.VMEM((TOPK * CCH, D), jnp.float32),
            pltpu.VMEM((TOPK * CCH, D), jnp.float32),
            pltpu.VMEM((CCH, D), jnp.float32),
            pltpu.VMEM((CCH, D), jnp.float32),
            pltpu.SemaphoreType.DMA,
            pltpu.SemaphoreType.DMA,
            pltpu.SemaphoreType.DMA,
            pltpu.SemaphoreType.DMA,
        ],
    )
    def k(eo_hbm, pos_hbm, out_hbm, idx_v, rows_a, rows_b, y_a, y_b,
          sem_a, sem_b, sem_ya, sem_yb):
        wid = lax.axis_index("s") * NC + lax.axis_index("c")
        base = wid * TOPK * CTOK
        nch = CTOK // CCH
        pltpu.sync_copy(pos_hbm.at[pl.ds(base, TOPK * CTOK)], idx_v)
        bufs = [rows_a, rows_b]
        sems = [sem_a, sem_b]
        ybufs = [y_a, y_b]
        ysems = [sem_ya, sem_yb]
        handles = [None] * nch
        yhandles = [None] * nch
        handles[0] = pltpu.async_copy(
            eo_hbm.at[idx_v.at[pl.ds(0, TOPK * CCH)]], bufs[0], sems[0])
        for c in range(nch):
            handles[c].wait()
            if c + 1 < nch:
                handles[c + 1] = pltpu.async_copy(
                    eo_hbm.at[idx_v.at[pl.ds((c + 1) * TOPK * CCH,
                                             TOPK * CCH)]],
                    bufs[(c + 1) % 2], sems[(c + 1) % 2])
            rows_v = bufs[c % 2]
            y_v = ybufs[c % 2]
            if c >= 2:
                yhandles[c - 2].wait()

            def pair_add(t, carry):
                def lane_add(j, carry2):
                    y_v[t, pl.ds(j * 16, 16)] = (
                        rows_v[2 * t, pl.ds(j * 16, 16)] +
                        rows_v[2 * t + 1, pl.ds(j * 16, 16)])
                    return carry2
                return lax.fori_loop(0, D // 16, lane_add, carry, unroll=8)
            lax.fori_loop(0, CCH, pair_add, 0)
            yhandles[c] = pltpu.async_copy(
                y_v, out_hbm.at[pl.ds(wid * CTOK + c * CCH, CCH)],
                ysems[c % 2])
        for c in range(max(0, nch - 2), nch):
            yhandles[c].wait()

    return k(eo, pos_flat)


def _shared_body(x_ref, sw1_ref, sb1_ref, sw3_ref, sb3_ref, sw2_ref, sb2_ref,
                 z_ref):
    x = x_ref[...]
    s1 = lax.dot_general(x, sw1_ref[...], (((1,), (1,)), ((), ())),
                         preferred_element_type=jnp.float32) + sb1_ref[...]
    s3 = lax.dot_general(x, sw3_ref[...], (((1,), (1,)), ((), ())),
                         preferred_element_type=jnp.float32) + sb3_ref[...]
    h = jnp.where(s1 >= 0, s1, 0.01 * s1) * s3
    z_ref[...] = lax.dot_general(h, sw2_ref[...], (((1,), (1,)), ((), ())),
                                 preferred_element_type=jnp.float32) + sb2_ref[...]


def _shared_ffn(x, sw1, sb1, sw3, sb3, sw2, sb2):
    return pl.pallas_call(
        _shared_body,
        grid=(N // SBLK,),
        in_specs=[
            pl.BlockSpec((SBLK, D), lambda b: (b, 0)),
            pl.BlockSpec((SI, D), lambda b: (0, 0)),
            pl.BlockSpec((1, SI), lambda b: (0, 0)),
            pl.BlockSpec((SI, D), lambda b: (0, 0)),
            pl.BlockSpec((1, SI), lambda b: (0, 0)),
            pl.BlockSpec((D, SI), lambda b: (0, 0)),
            pl.BlockSpec((1, D), lambda b: (0, 0)),
        ],
        out_specs=pl.BlockSpec((SBLK, D), lambda b: (b, 0)),
        out_shape=jax.ShapeDtypeStruct((N, D), jnp.float32),
    )(x, sw1, sb1.reshape(1, SI), sw3, sb3.reshape(1, SI), sw2,
      sb2.reshape(1, D))


def _final_body(y_ref, z_ref, ow_ref, ob_ref, out_ref):
    yz = y_ref[...] + z_ref[...]
    out_ref[...] = lax.dot_general(yz, ow_ref[...], (((1,), (1,)), ((), ())),
                                   preferred_element_type=jnp.float32) + ob_ref[...]


def _final(y, z, out_w, out_b):
    return pl.pallas_call(
        _final_body,
        grid=(N // FBLK,),
        in_specs=[
            pl.BlockSpec((FBLK, D), lambda b: (b, 0)),
            pl.BlockSpec((FBLK, D), lambda b: (b, 0)),
            pl.BlockSpec((OUT, D), lambda b: (0, 0)),
            pl.BlockSpec((1, OUT), lambda b: (0, 0)),
        ],
        out_specs=pl.BlockSpec((FBLK, OUT), lambda b: (b, 0)),
        out_shape=jax.ShapeDtypeStruct((N, OUT), jnp.float32),
    )(y, z, out_w, out_b.reshape(1, OUT))


@jax.jit
def _moe(x, gate_w, w1, b1, w2, b2, w3, b3, sw1, sb1, sw2, sb2, sw3, sb3,
         out_w, out_b):
    pos, wts, be, bv = _routing(x, gate_w)
    xs, wp = _sc_dispatch(
        x,
        pos[:, 0].reshape(NW, NCH, TCH), pos[:, 1].reshape(NW, NCH, TCH),
        wts[:, 0].reshape(NW, NCH, TCH), wts[:, 1].reshape(NW, NCH, TCH))
    z = _shared_ffn(x, sw1, sb1, sw3, sb3, sw2, sb2)
    eo = _grouped_gemm(xs, wp.reshape(P, 1), w1, b1, w3, b3, w2, b2,
                       be.reshape(NB), bv.reshape(NB))
    y = _sc_combine(eo, pos.reshape(TOPK * N))
    return _final(y, z, out_w, out_b)


def kernel(x, task_id, gate_w, W1, B1, W2, B2, W3, B3, sw1, sb1, sw2, sb2,
           sw3, sb3, out_w, out_b):
    xf = x.reshape(N, D)
    return _moe(xf, gate_w, W1, B1, W2, B2, W3, B3, sw1, sb1, sw2, sb2,
                sw3, sb3, out_w, out_b)


# speedup vs baseline: 2.2840x; 1.0042x over previous
import functools

import jax
import jax.numpy as jnp
from jax import lax
from jax.experimental import pallas as pl
from jax.experimental.pallas import tpu as pltpu
from jax.experimental.pallas import tpu_sc as plsc

E = 8
TOPK = 2
N = 2048
D = 1024
I_ = 1024
SI = 1024
OUT = 1024
ROUTE_SCALE = 1.0

BLK = 512
NB = (TOPK * N + E * (BLK - 1) + BLK - 1) // BLK
P = NB * BLK
SBLK = 2048
FBLK = 1024

NC = 2
NS = 16
NW = NC * NS
CTOK = N // NW
CCH = 16


def _routing_body(x_ref, gw_ref, pos_ref, w_ref, be_ref, bv_ref):
    x = x_ref[...]
    gw = gw_ref[...]
    s = lax.dot_general(x, gw, (((1,), (1,)), ((), ())),
                        preferred_element_type=jnp.float32)
    m = jnp.max(s, axis=1, keepdims=True)
    p = jnp.exp(s - m)
    p = p / jnp.sum(p, axis=1, keepdims=True)

    iota_e = lax.broadcasted_iota(jnp.int32, (N, E), 1)
    v1 = jnp.max(p, axis=1, keepdims=True)
    i1 = jnp.min(jnp.where(p == v1, iota_e, E), axis=1, keepdims=True)
    pm = jnp.where(iota_e == i1, -1.0, p)
    v2 = jnp.max(pm, axis=1, keepdims=True)
    i2 = jnp.min(jnp.where(pm == v2, iota_e, E), axis=1, keepdims=True)

    oh1 = (iota_e == i1).astype(jnp.float32)
    oh2 = (iota_e == i2).astype(jnp.float32)
    oh = oh1 + oh2
    c = jnp.sum(oh, axis=0, keepdims=True)
    ssz = jnp.floor((c + (BLK - 1)) * (1.0 / BLK)) * BLK
    u8s = (lax.broadcasted_iota(jnp.int32, (E, E), 0) <
           lax.broadcasted_iota(jnp.int32, (E, E), 1)).astype(jnp.float32)
    off = lax.dot_general(ssz, u8s, (((1,), (0,)), ((), ())),
                          precision=lax.Precision.HIGHEST,
                          preferred_element_type=jnp.float32)

    bstart = lax.broadcasted_iota(jnp.int32, (1, NB), 1).astype(jnp.float32) * BLK
    be = jnp.full((1, NB), float(E - 1), jnp.float32)
    for e in range(E):
        off_e = off[0:1, e:e + 1]
        end_e = off_e + ssz[0:1, e:e + 1]
        be = jnp.where((bstart >= off_e) & (bstart < end_e), float(e), be)
    total = off[0:1, E - 1:E] + ssz[0:1, E - 1:E]
    bv = (bstart < total)
    be_ref[...] = be.astype(jnp.int32)
    bv_ref[...] = bv.astype(jnp.int32)

    lns = (lax.broadcasted_iota(jnp.int32, (N, N), 1) <
           lax.broadcasted_iota(jnp.int32, (N, N), 0)).astype(jnp.float32)
    ex = lax.dot_general(lns, oh, (((1,), (0,)), ((), ())),
                         preferred_element_type=jnp.float32)
    pos0 = jnp.sum(oh1 * (off + ex), axis=1, keepdims=True)
    pos1 = jnp.sum(oh2 * (off + ex + oh1), axis=1, keepdims=True)

    pos_ref[...] = jnp.concatenate([pos0, pos1], axis=1).astype(jnp.int32)
    w_ref[...] = jnp.concatenate([v1, v2], axis=1) * ROUTE_SCALE


def _routing(x, gate_w):
    return pl.pallas_call(
        _routing_body,
        out_shape=[
            jax.ShapeDtypeStruct((N, TOPK), jnp.int32),
            jax.ShapeDtypeStruct((N, TOPK), jnp.float32),
            jax.ShapeDtypeStruct((1, NB), jnp.int32),
            jax.ShapeDtypeStruct((1, NB), jnp.int32),
        ],
    )(x, gate_w)


TCH = 64
NCH = CTOK // TCH


def _sc_dispatch(x, p0, p1, w0, w1):
    mesh = plsc.VectorSubcoreMesh(core_axis_name="c", subcore_axis_name="s",
                                  num_cores=NC)

    @functools.partial(
        pl.kernel, mesh=mesh,
        out_type=[
            jax.ShapeDtypeStruct((P, D), jnp.float32),
            jax.ShapeDtypeStruct((P,), jnp.float32),
        ],
        scratch_types=[
            pltpu.VMEM((CTOK, D), jnp.float32),
            pltpu.VMEM((NCH, TCH), jnp.int32),
            pltpu.VMEM((NCH, TCH), jnp.int32),
            pltpu.VMEM((NCH, TCH), jnp.float32),
            pltpu.VMEM((NCH, TCH), jnp.float32),
            pltpu.SemaphoreType.DMA,
            pltpu.SemaphoreType.DMA,
            pltpu.SemaphoreType.DMA,
            pltpu.SemaphoreType.DMA,
            pltpu.SemaphoreType.DMA,
        ],
    )
    def k(x_hbm, p0_hbm, p1_hbm, w0_hbm, w1_hbm, xs_hbm, wp_hbm,
          rows_v, i0_v, i1_v, w0_v, w1_v, sem, *sem_x):
        wid = lax.axis_index("s") * NC + lax.axis_index("c")
        xh = [pltpu.async_copy(
            x_hbm.at[pl.ds(wid * CTOK + c * TCH, TCH)],
            rows_v.at[pl.ds(c * TCH, TCH)], sem_x[c]) for c in range(NCH)]
        pltpu.sync_copy(p0_hbm.at[wid], i0_v)
        pltpu.sync_copy(p1_hbm.at[wid], i1_v)
        pltpu.sync_copy(w0_hbm.at[wid], w0_v)
        pltpu.sync_copy(w1_hbm.at[wid], w1_v)
        handles = []
        for c in range(NCH):
            xh[c].wait()
            src = rows_v.at[pl.ds(c * TCH, TCH)]
            handles.append(pltpu.async_copy(src, xs_hbm.at[i0_v.at[c]], sem))
            handles.append(pltpu.async_copy(src, xs_hbm.at[i1_v.at[c]], sem))
            handles.append(pltpu.async_copy(w0_v.at[c], wp_hbm.at[i0_v.at[c]],
                                            sem))
            handles.append(pltpu.async_copy(w1_v.at[c], wp_hbm.at[i1_v.at[c]],
                                            sem))
        for h in handles:
            h.wait()

    return k(x, p0, p1, w0, w1)


def _gemm_body(be_ref, bv_ref, xs_ref, wp_ref, w1_ref, b1_ref, w3_ref,
               b3_ref, w2_ref, b2_ref, eo_ref):
    b = pl.program_id(0)

    @pl.when(bv_ref[b] == 1)
    def _():
        xs = xs_ref[...]
        h1 = lax.dot_general(xs, w1_ref[0], (((1,), (1,)), ((), ())),
                             preferred_element_type=jnp.float32) + b1_ref[0]
        h3 = lax.dot_general(xs, w3_ref[0], (((1,), (1,)), ((), ())),
                             preferred_element_type=jnp.float32) + b3_ref[0]
        h = jnp.where(h1 >= 0, h1, 0.01 * h1) * h3
        eo = lax.dot_general(h, w2_ref[0], (((1,), (1,)), ((), ())),
                             preferred_element_type=jnp.float32) + b2_ref[0]
        eo_ref[...] = eo * wp_ref[...]

    @pl.when(bv_ref[b] == 0)
    def _():
        eo_ref[...] = jnp.zeros((BLK, D), jnp.float32)


def _grouped_gemm(xs, wp, w1, b1, w3, b3, w2, b2, be, bv):
    grid_spec = pltpu.PrefetchScalarGridSpec(
        num_scalar_prefetch=2,
        grid=(NB,),
        in_specs=[
            pl.BlockSpec((BLK, D), lambda b, be, bv: (b, 0)),
            pl.BlockSpec((BLK, 1), lambda b, be, bv: (b, 0)),
            pl.BlockSpec((1, I_, D), lambda b, be, bv: (be[b], 0, 0)),
            pl.BlockSpec((1, 1, I_), lambda b, be, bv: (be[b], 0, 0)),
            pl.BlockSpec((1, I_, D), lambda b, be, bv: (be[b], 0, 0)),
            pl.BlockSpec((1, 1, I_), lambda b, be, bv: (be[b], 0, 0)),
            pl.BlockSpec((1, D, I_), lambda b, be, bv: (be[b], 0, 0)),
            pl.BlockSpec((1, 1, D), lambda b, be, bv: (be[b], 0, 0)),
        ],
        out_specs=pl.BlockSpec((BLK, D), lambda b, be, bv: (b, 0)),
    )
    return pl.pallas_call(
        _gemm_body,
        grid_spec=grid_spec,
        out_shape=jax.ShapeDtypeStruct((P, D), jnp.float32),
    )(be, bv, xs, wp, w1, b1.reshape(E, 1, I_), w3, b3.reshape(E, 1, I_),
      w2, b2.reshape(E, 1, D))


def _sc_combine(eo, pos_flat):
    mesh = plsc.VectorSubcoreMesh(core_axis_name="c", subcore_axis_name="s",
                                  num_cores=NC)

    @functools.partial(
        pl.kernel, mesh=mesh,
        out_type=jax.ShapeDtypeStruct((N, D), jnp.float32),
        scratch_types=[
            pltpu.VMEM((TOPK * CTOK,), jnp.int32),
            pltpu.VMEM((TOPK * CCH, D), jnp.float32),
            pltpu.VMEM((TOPK * CCH, D), jnp.float32),
            pltpu.VMEM((CCH, D), jnp.float32),
            pltpu.VMEM((CCH, D), jnp.float32),
            pltpu.SemaphoreType.DMA,
            pltpu.SemaphoreType.DMA,
            pltpu.SemaphoreType.DMA,
            pltpu.SemaphoreType.DMA,
        ],
    )
    def k(eo_hbm, pos_hbm, out_hbm, idx_v, rows_a, rows_b, y_a, y_b,
          sem_a, sem_b, sem_ya, sem_yb):
        wid = lax.axis_index("s") * NC + lax.axis_index("c")
        base = wid * TOPK * CTOK
        nch = CTOK // CCH
        pltpu.sync_copy(pos_hbm.at[pl.ds(base, TOPK * CTOK)], idx_v)
        bufs = [rows_a, rows_b]
        sems = [sem_a, sem_b]
        ybufs = [y_a, y_b]
        ysems = [sem_ya, sem_yb]
        handles = [None] * nch
        yhandles = [None] * nch
        handles[0] = pltpu.async_copy(
            eo_hbm.at[idx_v.at[pl.ds(0, TOPK * CCH)]], bufs[0], sems[0])
        for c in range(nch):
            handles[c].wait()
            if c + 1 < nch:
                handles[c + 1] = pltpu.async_copy(
                    eo_hbm.at[idx_v.at[pl.ds((c + 1) * TOPK * CCH,
                                             TOPK * CCH)]],
                    bufs[(c + 1) % 2], sems[(c + 1) % 2])
            rows_v = bufs[c % 2]
            y_v = ybufs[c % 2]
            if c >= 2:
                yhandles[c - 2].wait()

            def pair_add(t, carry):
                def lane_add(j, carry2):
                    y_v[t, pl.ds(j * 16, 16)] = (
                        rows_v[2 * t, pl.ds(j * 16, 16)] +
                        rows_v[2 * t + 1, pl.ds(j * 16, 16)])
                    return carry2
                return lax.fori_loop(0, D // 16, lane_add, carry, unroll=8)
            lax.fori_loop(0, CCH, pair_add, 0)
            yhandles[c] = pltpu.async_copy(
                y_v, out_hbm.at[pl.ds(wid * CTOK + c * CCH, CCH)],
                ysems[c % 2])
        for c in range(max(0, nch - 2), nch):
            yhandles[c].wait()

    return k(eo, pos_flat)


def _shared_body(x_ref, sw1_ref, sb1_ref, sw3_ref, sb3_ref, sw2_ref, sb2_ref,
                 z_ref):
    x = x_ref[...]
    s1 = lax.dot_general(x, sw1_ref[...], (((1,), (1,)), ((), ())),
                         preferred_element_type=jnp.float32) + sb1_ref[...]
    s3 = lax.dot_general(x, sw3_ref[...], (((1,), (1,)), ((), ())),
                         preferred_element_type=jnp.float32) + sb3_ref[...]
    h = jnp.where(s1 >= 0, s1, 0.01 * s1) * s3
    z_ref[...] = lax.dot_general(h, sw2_ref[...], (((1,), (1,)), ((), ())),
                                 preferred_element_type=jnp.float32) + sb2_ref[...]


def _shared_ffn(x, sw1, sb1, sw3, sb3, sw2, sb2):
    return pl.pallas_call(
        _shared_body,
        grid=(N // SBLK,),
        in_specs=[
            pl.BlockSpec((SBLK, D), lambda b: (b, 0)),
            pl.BlockSpec((SI, D), lambda b: (0, 0)),
            pl.BlockSpec((1, SI), lambda b: (0, 0)),
            pl.BlockSpec((SI, D), lambda b: (0, 0)),
            pl.BlockSpec((1, SI), lambda b: (0, 0)),
            pl.BlockSpec((D, SI), lambda b: (0, 0)),
            pl.BlockSpec((1, D), lambda b: (0, 0)),
        ],
        out_specs=pl.BlockSpec((SBLK, D), lambda b: (b, 0)),
        out_shape=jax.ShapeDtypeStruct((N, D), jnp.float32),
    )(x, sw1, sb1.reshape(1, SI), sw3, sb3.reshape(1, SI), sw2,
      sb2.reshape(1, D))


def _final_body(y_ref, z_ref, ow_ref, ob_ref, out_ref):
    yz = y_ref[...] + z_ref[...]
    out_ref[...] = lax.dot_general(yz, ow_ref[...], (((1,), (1,)), ((), ())),
                                   preferred_element_type=jnp.float32) + ob_ref[...]


def _final(y, z, out_w, out_b):
    return pl.pallas_call(
        _final_body,
        grid=(N // FBLK,),
        in_specs=[
            pl.BlockSpec((FBLK, D), lambda b: (b, 0)),
            pl.BlockSpec((FBLK, D), lambda b: (b, 0)),
            pl.BlockSpec((OUT, D), lambda b: (0, 0)),
            pl.BlockSpec((1, OUT), lambda b: (0, 0)),
        ],
        out_specs=pl.BlockSpec((FBLK, OUT), lambda b: (b, 0)),
        out_shape=jax.ShapeDtypeStruct((N, OUT), jnp.float32),
    )(y, z, out_w, out_b.reshape(1, OUT))


@jax.jit
def _moe(x, gate_w, w1, b1, w2, b2, w3, b3, sw1, sb1, sw2, sb2, sw3, sb3,
         out_w, out_b):
    pos, wts, be, bv = _routing(x, gate_w)
    xs, wp = _sc_dispatch(
        x,
        pos[:, 0].reshape(NW, NCH, TCH), pos[:, 1].reshape(NW, NCH, TCH),
        wts[:, 0].reshape(NW, NCH, TCH), wts[:, 1].reshape(NW, NCH, TCH))
    z = _shared_ffn(x, sw1, sb1, sw3, sb3, sw2, sb2)
    eo = _grouped_gemm(xs, wp.reshape(P, 1), w1, b1, w3, b3, w2, b2,
                       be.reshape(NB), bv.reshape(NB))
    y = _sc_combine(eo, pos.reshape(TOPK * N))
    return _final(y, z, out_w, out_b)


def kernel(x, task_id, gate_w, W1, B1, W2, B2, W3, B3, sw1, sb1, sw2, sb2,
           sw3, sb3, out_w, out_b):
    xf = x.reshape(N, D)
    return _moe(xf, gate_w, W1, B1, W2, B2, W3, B3, sw1, sb1, sw2, sb2,
                sw3, sb3, out_w, out_b)
